# baseline (speedup 1.0000x reference)
import functools

import jax
import jax.numpy as jnp
from jax import lax
from jax.experimental import pallas as pl
from jax.experimental.pallas import tpu as pltpu
from jax.experimental.pallas import tpu_sc as plsc

N_NODES = 10000
N_EDGES = 160000
B = 16
Q = 20
K_TOP = 256
D_FEAT = 256
PROP_THRESHOLD = 0.5
ATTEN_COEF = 0.25

QP = 32
EH = N_EDGES // 2
CH = 4000
NCHUNK = EH // CH
CBUF = 272
NCAND = 4 * CBUF

_mesh = plsc.VectorSubcoreMesh(core_axis_name="c", subcore_axis_name="s")
_sc_params = pltpu.CompilerParams(needs_layout_passes=False)


def _splat_i32(x):
    return jnp.zeros((16,), jnp.int32) + x


def _splat_f32(x):
    return jnp.zeros((16,), jnp.float32) + x


@functools.partial(
    pl.kernel,
    out_type=jax.ShapeDtypeStruct((B * N_EDGES,), jnp.float32),
    mesh=_mesh,
    compiler_params=_sc_params,
    scratch_types=[
        pltpu.VMEM((QP,), jnp.int32),
        pltpu.VMEM((QP,), jnp.float32),
        pltpu.VMEM((QP,), jnp.float32),
        pltpu.VMEM((QP,), jnp.float32),
        pltpu.VMEM((N_NODES,), jnp.float32),
        pltpu.VMEM((N_NODES,), jnp.float32),
        pltpu.VMEM((CH,), jnp.int32),
        pltpu.VMEM((CH,), jnp.int32),
        pltpu.VMEM((CH,), jnp.float32),
        pltpu.VMEM((CH,), jnp.int32),
        pltpu.VMEM((CH,), jnp.int32),
        pltpu.VMEM((CH,), jnp.float32),
        pltpu.VMEM((CH,), jnp.float32),
        pltpu.SemaphoreType.DMA,
        pltpu.SemaphoreType.DMA,
    ],
)
def _k23_graph(q_hbm, a_hbm, w_hbm, src_hbm, dst_hbm, init_hbm, graph_hbm,
               qv, av, wv, gv, Ct, At, sbuf0, dbuf0, ibuf0,
               sbuf1, dbuf1, ibuf1, obuf, sem0, sem1):
    b = lax.axis_index("s")
    c = lax.axis_index("c")
    pltpu.sync_copy(q_hbm.at[pl.ds(pl.multiple_of(b * QP, 8), QP)], qv)
    pltpu.sync_copy(a_hbm.at[pl.ds(pl.multiple_of(b * QP, 8), QP)], av)
    pltpu.sync_copy(w_hbm, wv)

    for t in range(2):
        sl = pl.ds(t * 16, 16)
        x = av[sl] * wv[sl]
        imp = 1.0 / (1.0 + jnp.exp(-x))
        g = jnp.where(imp >= PROP_THRESHOLD, imp, 0.0)
        if t == 1:
            lane = lax.broadcasted_iota(jnp.int32, (16,), 0)
            g = jnp.where(lane < (Q - 16), g, 0.0)
        gv[sl] = g

    def _zero(i, _):
        Ct[pl.ds(i * 16, 16)] = jnp.zeros((16,), jnp.float32)
        return 0
    lax.fori_loop(0, N_NODES // 16, _zero, 0)

    for t in range(2):
        sl = pl.ds(t * 16, 16)
        plsc.addupdate_scatter(Ct, [qv[sl]], gv[sl])

    atts = []
    n_att = jnp.int32(0)
    for t in range(2):
        sl = pl.ds(t * 16, 16)
        qch = qv[sl]
        v = jnp.zeros((16,), jnp.float32)
        for j in range(Q):
            qs = plsc.load_gather(qv, [jnp.full((16,), j, jnp.int32)])
            gs = plsc.load_gather(gv, [jnp.full((16,), j, jnp.int32)])
            v = v + jnp.where(qch == qs, gs, 0.0)
        a = ATTEN_COEF * v
        a = jnp.where(a >= PROP_THRESHOLD, a, 0.0)
        atts.append(a)
        n_att = n_att + jnp.max(plsc.all_reduce_population_count(a > 0.0))

    @pl.when(n_att > 0)
    def _slow_hop():
        def _zeroA(i, _):
            At[pl.ds(i * 16, 16)] = jnp.zeros((16,), jnp.float32)
            return 0
        lax.fori_loop(0, N_NODES // 16, _zeroA, 0)
        for t in range(2):
            plsc.store_scatter(At, [qv[pl.ds(t * 16, 16)]], atts[t])

        def _chunk(ci, _):
            base = pl.multiple_of(ci * CH, 8)
            pltpu.sync_copy(src_hbm.at[pl.ds(base, CH)], sbuf0)
            pltpu.sync_copy(dst_hbm.at[pl.ds(base, CH)], dbuf0)

            def _vec(k, __):
                sl = pl.ds(k * 16, 16)
                sv = sbuf0[sl]
                dv = dbuf0[sl]
                a_s = plsc.load_gather(At, [sv])
                a_d = plsc.load_gather(At, [dv])
                plsc.addupdate_scatter(Ct, [dv], a_s)
                plsc.addupdate_scatter(Ct, [sv], a_d)
                return 0
            lax.fori_loop(0, CH // 16, _vec, 0)
            return 0
        lax.fori_loop(0, N_EDGES // CH, _chunk, 0)

    bufs = ((sbuf0, dbuf0, ibuf0, sem0), (sbuf1, dbuf1, ibuf1, sem1))

    def _in_slices(ci):
        base = pl.multiple_of(c * EH + ci * CH, 8)
        return (src_hbm.at[pl.ds(base, CH)], dst_hbm.at[pl.ds(base, CH)],
                init_hbm.at[pl.ds(base, CH)])

    def _start_in(ci, bs):
        s0, s1, s2 = _in_slices(ci)
        pltpu.async_copy(s0, bs[0], bs[3])
        pltpu.async_copy(s1, bs[1], bs[3])
        pltpu.async_copy(s2, bs[2], bs[3])

    def _wait_in(ci, bs):
        s0, s1, s2 = _in_slices(ci)
        pltpu.make_async_copy(s0, bs[0], bs[3]).wait()
        pltpu.make_async_copy(s1, bs[1], bs[3]).wait()
        pltpu.make_async_copy(s2, bs[2], bs[3]).wait()

    def _compute(ci, bs):
        sb, db, ib = bs[0], bs[1], bs[2]

        def _vec(k, __):
            for u in range(2):
                sl = pl.ds((k * 2 + u) * 16, 16)
                cs = plsc.load_gather(Ct, [sb[sl]])
                cd = plsc.load_gather(Ct, [db[sl]])
                obuf[sl] = ib[sl] + cs + cd
            return 0
        lax.fori_loop(0, CH // 32, _vec, 0)
        gbase = pl.multiple_of(b * N_EDGES + c * EH + ci * CH, 8)
        pltpu.sync_copy(obuf, graph_hbm.at[pl.ds(gbase, CH)])

    _start_in(jnp.int32(0), bufs[0])

    def _pair(pi, _):
        ci0 = pi * 2
        _wait_in(ci0, bufs[0])
        _start_in(ci0 + 1, bufs[1])
        _compute(ci0, bufs[0])
        _wait_in(ci0 + 1, bufs[1])

        @pl.when(pi < NCHUNK // 2 - 1)
        def _():
            _start_in(ci0 + 2, bufs[0])
        _compute(ci0 + 1, bufs[1])
        return 0
    lax.fori_loop(0, NCHUNK // 2, _pair, 0)


_CB = 6400
_NB = N_EDGES // _CB


def _k4_body(graph_ref, T_ref, ngt_ref, need_ref):
    def _mask_sum(bv, test):
        m = jnp.where(bv >= test, jnp.int32(1), jnp.int32(0))
        return jnp.sum(m.reshape(B, _CB // 128, 128), axis=1)

    def count_ge(test):
        def blk(k, acc):
            v = graph_ref[:, pl.ds(k * _CB, _CB)]
            bv = lax.bitcast_convert_type(v, jnp.int32)
            return acc + _mask_sum(bv, test)
        acc = lax.fori_loop(0, _NB, blk, jnp.zeros((B, 128), jnp.int32))
        return jnp.sum(acc, axis=1, keepdims=True)

    cand = jnp.where(count_ge(jnp.full((B, 1), 1 << 30, jnp.int32)) >= K_TOP,
                     jnp.full((B, 1), 1 << 30, jnp.int32),
                     jnp.zeros((B, 1), jnp.int32))

    def radix_step(i, cand):
        shift = 28 - 2 * i
        t1 = cand | (1 << shift)
        t2 = cand | (2 << shift)
        t3 = cand | (3 << shift)

        def blk(k, accs):
            a1, a2, a3 = accs
            v = graph_ref[:, pl.ds(k * _CB, _CB)]
            bv = lax.bitcast_convert_type(v, jnp.int32)
            a1 = a1 + _mask_sum(bv, t1)
            a2 = a2 + _mask_sum(bv, t2)
            a3 = a3 + _mask_sum(bv, t3)
            return (a1, a2, a3)
        z = jnp.zeros((B, 128), jnp.int32)
        a1, a2, a3 = lax.fori_loop(0, _NB, blk, (z, z, z))
        c1 = jnp.sum(a1, axis=1, keepdims=True)
        c2 = jnp.sum(a2, axis=1, keepdims=True)
        c3 = jnp.sum(a3, axis=1, keepdims=True)
        return jnp.where(
            c3 >= K_TOP, t3,
            jnp.where(c2 >= K_TOP, t2, jnp.where(c1 >= K_TOP, t1, cand)))

    cand = lax.fori_loop(0, 15, radix_step, cand)
    ngt = count_ge(cand + 1)
    T_ref[...] = lax.bitcast_convert_type(cand, jnp.float32)
    ngt_ref[...] = ngt
    need_ref[...] = K_TOP - ngt


_k4_threshold = pl.pallas_call(
    _k4_body,
    out_shape=(
        jax.ShapeDtypeStruct((B, 1), jnp.float32),
        jax.ShapeDtypeStruct((B, 1), jnp.int32),
        jax.ShapeDtypeStruct((B, 1), jnp.int32),
    ),
)


_K5CH = 4000
_K5NCH = EH // _K5CH


@functools.partial(
    pl.kernel,
    out_type=(
        jax.ShapeDtypeStruct((B * 2 * CBUF,), jnp.float32),
        jax.ShapeDtypeStruct((B * 2 * CBUF,), jnp.int32),
        jax.ShapeDtypeStruct((B * 2 * CBUF,), jnp.float32),
        jax.ShapeDtypeStruct((B * 2 * CBUF,), jnp.int32),
        jax.ShapeDtypeStruct((B * 2 * 2 * 16,), jnp.int32),
    ),
    mesh=_mesh,
    compiler_params=_sc_params,
    scratch_types=[
        pltpu.VMEM((16,), jnp.float32),
        pltpu.VMEM((16,), jnp.int32),
        pltpu.VMEM((_K5CH,), jnp.float32),
        pltpu.VMEM((CBUF,), jnp.float32),
        pltpu.VMEM((CBUF,), jnp.int32),
        pltpu.VMEM((CBUF,), jnp.float32),
        pltpu.VMEM((CBUF,), jnp.int32),
        pltpu.VMEM((32,), jnp.int32),
    ],
)
def _k5_compact(graph_hbm, t_hbm, need_hbm,
                gtv_hbm, gti_hbm, eqv_hbm, eqi_hbm, ngt_hbm,
                Tv, needv, cbuf, gvb, gib, evb, eib, nsplat):
    b = lax.axis_index("s")
    c = lax.axis_index("c")
    pltpu.sync_copy(t_hbm, Tv)
    pltpu.sync_copy(need_hbm, needv)
    tb = plsc.load_gather(Tv, [_splat_i32(b)])
    nb = plsc.load_gather(needv, [_splat_i32(b)])

    def _zero(i, _):
        sl = pl.ds(i * 16, 16)
        gvb[sl] = jnp.zeros((16,), jnp.float32)
        gib[sl] = jnp.zeros((16,), jnp.int32)
        evb[sl] = jnp.zeros((16,), jnp.float32)
        eib[sl] = jnp.zeros((16,), jnp.int32)
        return 0
    lax.fori_loop(0, CBUF // 16, _zero, 0)

    iota16 = lax.broadcasted_iota(jnp.int32, (16,), 0)
    ebase = c * EH

    def _chunk(ci, carry):
        gb = pl.multiple_of(b * N_EDGES + ebase + ci * _K5CH, 8)
        pltpu.sync_copy(graph_hbm.at[pl.ds(gb, _K5CH)], cbuf)

        def _grp(k, cr):
            vs = [cbuf[pl.ds((k * 5 + i) * 16, 16)] for i in range(5)]
            pc = plsc.all_reduce_population_count(vs[0] >= tb)
            for i in range(1, 5):
                pc = pc + plsc.all_reduce_population_count(vs[i] >= tb)
            hits = pc[0]

            def _slow(cr2):
                for i in range(5):
                    v = vs[i]
                    ngt, mst, etot = cr2
                    m_gt = v > tb
                    m_eq = v == tb
                    gidx = (_splat_i32(ebase + ci * _K5CH + (k * 5 + i) * 16)
                            + iota16)
                    eqrank = plsc.cumsum(m_eq.astype(jnp.int32))
                    take = m_eq & ((_splat_i32(etot) + eqrank) <= nb)
                    plsc.store_compressed(gvb.at[pl.ds(ngt, 16)], v,
                                          mask=m_gt)
                    plsc.store_compressed(gib.at[pl.ds(ngt, 16)], gidx,
                                          mask=m_gt)
                    plsc.store_compressed(evb.at[pl.ds(mst, 16)], v,
                                          mask=take)
                    plsc.store_compressed(eib.at[pl.ds(mst, 16)], gidx,
                                          mask=take)
                    ngt = ngt + plsc.all_reduce_population_count(m_gt)[0]
                    mst = mst + plsc.all_reduce_population_count(take)[0]
                    etot = etot + plsc.all_reduce_population_count(m_eq)[0]
                    cr2 = (ngt, mst, etot)
                return cr2

            return lax.cond(hits > 0, _slow, lambda cr2: cr2, cr)
        return lax.fori_loop(0, _K5CH // 80, _grp, carry)

    ngt, mst, etot = lax.fori_loop(
        0, _K5NCH, _chunk, (jnp.int32(0), jnp.int32(0), jnp.int32(0)))
    nsplat[pl.ds(0, 16)] = _splat_i32(ngt)
    nsplat[pl.ds(16, 16)] = _splat_i32(etot)
    ob = pl.multiple_of((b * 2 + c) * CBUF, 8)
    pltpu.sync_copy(gvb, gtv_hbm.at[pl.ds(ob, CBUF)])
    pltpu.sync_copy(gib, gti_hbm.at[pl.ds(ob, CBUF)])
    pltpu.sync_copy(evb, eqv_hbm.at[pl.ds(ob, CBUF)])
    pltpu.sync_copy(eib, eqi_hbm.at[pl.ds(ob, CBUF)])
    pltpu.sync_copy(nsplat,
                    ngt_hbm.at[pl.ds(pl.multiple_of((b * 2 + c) * 32, 8), 32)])


def _k6_body(vrow_ref, vcol_ref, irow_ref, icol_ref, ngt_ref, delta_ref,
             w_ref, if_ref):
    vrow = vrow_ref[0]
    vcol = vcol_ref[0]
    irow = irow_ref[0]
    icol = icol_ref[0]
    cnts = ngt_ref[0]
    n0 = jnp.max(cnts[:, 0:16])
    e0 = jnp.max(cnts[:, 16:32])
    n1 = jnp.max(cnts[:, 32:48])
    e1 = jnp.max(cnts[:, 48:64])
    need = K_TOP - n0 - n1
    m0 = jnp.minimum(need, e0)
    m1 = need - e0

    one = jnp.int32(1)
    zero = jnp.int32(0)

    def _valid(pos):
        r0 = jnp.where(pos < n0, one, zero)
        r1 = jnp.where(pos - CBUF < n1, one, zero)
        r2 = jnp.where(pos - 2 * CBUF < m0, one, zero)
        r3 = jnp.where(pos - 3 * CBUF < m1, one, zero)
        lo = jnp.where(pos < CBUF, r0, r1)
        hi = jnp.where(pos < 3 * CBUF, r2, r3)
        return jnp.where(pos < 2 * CBUF, lo, hi)

    lane = lax.broadcasted_iota(jnp.int32, (1, NCAND), 1)
    valid_row = _valid(lane)
    sub = lax.broadcasted_iota(jnp.int32, (NCAND, 1), 0)
    valid_col = _valid(sub)

    onef = jnp.float32(1.0)
    zerof = jnp.float32(0.0)
    gt_f = jnp.where(vrow > vcol, onef, zerof)
    eq_f = jnp.where(vrow == vcol, onef, zerof)
    lt_f = jnp.where(irow < icol, onef, zerof)
    beats = (gt_f + eq_f * lt_f) * valid_row.astype(jnp.float32)
    posf = jax.lax.dot_general(
        beats, jnp.ones((NCAND, 1), jnp.float32),
        (((1,), (0,)), ((), ())),
        preferred_element_type=jnp.float32,
        precision=jax.lax.Precision.HIGHEST)
    pos = posf.astype(jnp.int32)

    kk = lax.broadcasted_iota(jnp.int32, (1, K_TOP), 1)
    oh = jnp.where(pos == kk, one, zero) * valid_col
    ohf = oh.astype(jnp.float32)
    w = jnp.sum(vcol * ohf, axis=0, keepdims=True)
    idx = jnp.sum(icol * ohf, axis=0, keepdims=True)
    w_ref[0] = w + delta_ref[0, 0, 0]
    if_ref[0] = idx


_k6_sort = pl.pallas_call(
    _k6_body,
    grid=(B,),
    in_specs=[
        pl.BlockSpec((1, 1, NCAND), lambda b: (b, 0, 0)),
        pl.BlockSpec((1, NCAND, 1), lambda b: (b, 0, 0)),
        pl.BlockSpec((1, 1, NCAND), lambda b: (b, 0, 0)),
        pl.BlockSpec((1, NCAND, 1), lambda b: (b, 0, 0)),
        pl.BlockSpec((1, 1, 64), lambda b: (b, 0, 0)),
        pl.BlockSpec((1, 1, 1), lambda b: (0, 0, 0)),
    ],
    out_specs=(
        pl.BlockSpec((1, 1, K_TOP), lambda b: (b, 0, 0)),
        pl.BlockSpec((1, 1, K_TOP), lambda b: (b, 0, 0)),
    ),
    out_shape=(
        jax.ShapeDtypeStruct((B, 1, K_TOP), jnp.float32),
        jax.ShapeDtypeStruct((B, 1, K_TOP), jnp.float32),
    ),
)


_ROWS_PER_W = (B * K_TOP) // 32


@functools.partial(
    pl.kernel,
    out_type=jax.ShapeDtypeStruct((B * K_TOP, D_FEAT), jnp.float32),
    mesh=_mesh,
    compiler_params=_sc_params,
    scratch_types=[
        pltpu.VMEM((_ROWS_PER_W,), jnp.int32),
        pltpu.VMEM((_ROWS_PER_W,), jnp.int32),
        pltpu.VMEM((_ROWS_PER_W,), jnp.float32),
        pltpu.VMEM((_ROWS_PER_W, D_FEAT), jnp.float32),
        pltpu.SemaphoreType.DMA,
    ],
)
def _k7_gather(topidx_hbm, topw_hbm, src_hbm, emb_hbm, out_hbm,
               ev, nv, wv, rows, sem):
    b = lax.axis_index("s")
    c = lax.axis_index("c")
    w = b * 2 + c
    base = pl.multiple_of(w * _ROWS_PER_W, 8)
    pltpu.sync_copy(topidx_hbm.at[pl.ds(base, _ROWS_PER_W)], ev)
    pltpu.sync_copy(topw_hbm.at[pl.ds(base, _ROWS_PER_W)], wv)
    pltpu.async_copy(src_hbm.at[ev], nv, sem).wait()
    pltpu.async_copy(emb_hbm.at[nv], rows, sem).wait()

    def _scale_row(i, _):
        ws = plsc.load_gather(wv, [_splat_i32(i)])
        for ch in range(D_FEAT // 16):
            sl = pl.ds(ch * 16, 16)
            rows[i, sl] = rows[i, sl] * ws
        return 0
    lax.fori_loop(0, _ROWS_PER_W, _scale_row, 0)
    pltpu.sync_copy(rows, out_hbm.at[pl.ds(base, _ROWS_PER_W)])


def kernel(list_questions, attention_question, edge_index, num_max_nodes,
           init_graph_tensor, node_embedding, w_importance):
    f32 = jnp.float32
    i32 = jnp.int32
    src = edge_index[0].astype(i32)
    dst = edge_index[1].astype(i32)

    qpad = jnp.zeros((B, QP), i32).at[:, :Q].set(
        list_questions.astype(i32)).reshape(B * QP)
    apad = jnp.zeros((B, QP), f32).at[:, :Q].set(
        attention_question).reshape(B * QP)
    wpad = jnp.zeros((QP,), f32).at[:Q].set(w_importance)

    graph = _k23_graph(qpad, apad, wpad, src, dst, init_graph_tensor)
    t_b, ngt_b, need_b = _k4_threshold(graph.reshape(B, N_EDGES))
    gtv, gti, eqv, eqi, ngt = _k5_compact(
        graph, t_b.reshape(B), need_b.reshape(B))
    gtv = gtv.reshape(B, 2 * CBUF)
    gti = gti.reshape(B, 2 * CBUF)
    eqv = eqv.reshape(B, 2 * CBUF)
    eqi = eqi.reshape(B, 2 * CBUF)
    ngt = ngt.reshape(B, 64)

    cat_v = jnp.concatenate([gtv, eqv], axis=1)
    cat_i = jnp.concatenate([gti, eqi], axis=1).astype(f32)
    delta = jnp.asarray(num_max_nodes - K_TOP, f32).reshape(1, 1, 1)
    top_w, top_if = _k6_sort(
        cat_v[:, None, :], cat_v[:, :, None],
        cat_i[:, None, :], cat_i[:, :, None],
        ngt[:, None, :], delta)

    top_idx = top_if.reshape(B * K_TOP).astype(i32)
    out = _k7_gather(top_idx, top_w.reshape(B * K_TOP), src, node_embedding)
    return out.reshape(B, K_TOP, D_FEAT)


# speedup vs baseline: 1.3925x; 1.3925x over previous
import functools

import jax
import jax.numpy as jnp
from jax import lax
from jax.experimental import pallas as pl
from jax.experimental.pallas import tpu as pltpu
from jax.experimental.pallas import tpu_sc as plsc

N_NODES = 10000
N_EDGES = 160000
B = 16
Q = 20
K_TOP = 256
D_FEAT = 256
PROP_THRESHOLD = 0.5
ATTEN_COEF = 0.25

QP = 32
EH = N_EDGES // 2
CH = 4000
NCHUNK = EH // CH
CBUF = 272
NCAND = 4 * CBUF

_mesh = plsc.VectorSubcoreMesh(core_axis_name="c", subcore_axis_name="s")
_sc_params = pltpu.CompilerParams(needs_layout_passes=False)


def _splat_i32(x):
    return jnp.zeros((16,), jnp.int32) + x


def _splat_f32(x):
    return jnp.zeros((16,), jnp.float32) + x


@functools.partial(
    pl.kernel,
    out_type=jax.ShapeDtypeStruct((B * N_EDGES,), jnp.float32),
    mesh=_mesh,
    compiler_params=_sc_params,
    scratch_types=[
        pltpu.VMEM((QP,), jnp.int32),
        pltpu.VMEM((QP,), jnp.float32),
        pltpu.VMEM((QP,), jnp.float32),
        pltpu.VMEM((QP,), jnp.float32),
        pltpu.VMEM((N_NODES,), jnp.float32),
        pltpu.VMEM((N_NODES,), jnp.float32),
        pltpu.VMEM((CH,), jnp.int32),
        pltpu.VMEM((CH,), jnp.int32),
        pltpu.VMEM((CH,), jnp.float32),
        pltpu.VMEM((CH,), jnp.int32),
        pltpu.VMEM((CH,), jnp.int32),
        pltpu.VMEM((CH,), jnp.float32),
        pltpu.VMEM((CH,), jnp.float32),
        pltpu.SemaphoreType.DMA,
        pltpu.SemaphoreType.DMA,
    ],
)
def _k23_graph(q_hbm, a_hbm, w_hbm, src_hbm, dst_hbm, init_hbm, graph_hbm,
               qv, av, wv, gv, Ct, At, sbuf0, dbuf0, ibuf0,
               sbuf1, dbuf1, ibuf1, obuf, sem0, sem1):
    b = lax.axis_index("s")
    c = lax.axis_index("c")
    pltpu.sync_copy(q_hbm.at[pl.ds(pl.multiple_of(b * QP, 8), QP)], qv)
    pltpu.sync_copy(a_hbm.at[pl.ds(pl.multiple_of(b * QP, 8), QP)], av)
    pltpu.sync_copy(w_hbm, wv)

    for t in range(2):
        sl = pl.ds(t * 16, 16)
        x = av[sl] * wv[sl]
        imp = 1.0 / (1.0 + jnp.exp(-x))
        g = jnp.where(imp >= PROP_THRESHOLD, imp, 0.0)
        if t == 1:
            lane = lax.broadcasted_iota(jnp.int32, (16,), 0)
            g = jnp.where(lane < (Q - 16), g, 0.0)
        gv[sl] = g

    def _zero(i, _):
        Ct[pl.ds(i * 16, 16)] = jnp.zeros((16,), jnp.float32)
        return 0
    lax.fori_loop(0, N_NODES // 16, _zero, 0)

    for t in range(2):
        sl = pl.ds(t * 16, 16)
        plsc.addupdate_scatter(Ct, [qv[sl]], gv[sl])

    atts = []
    n_att = jnp.int32(0)
    for t in range(2):
        sl = pl.ds(t * 16, 16)
        qch = qv[sl]
        v = jnp.zeros((16,), jnp.float32)
        for j in range(Q):
            qs = plsc.load_gather(qv, [jnp.full((16,), j, jnp.int32)])
            gs = plsc.load_gather(gv, [jnp.full((16,), j, jnp.int32)])
            v = v + jnp.where(qch == qs, gs, 0.0)
        a = ATTEN_COEF * v
        a = jnp.where(a >= PROP_THRESHOLD, a, 0.0)
        atts.append(a)
        n_att = n_att + jnp.max(plsc.all_reduce_population_count(a > 0.0))

    @pl.when(n_att > 0)
    def _slow_hop():
        def _zeroA(i, _):
            At[pl.ds(i * 16, 16)] = jnp.zeros((16,), jnp.float32)
            return 0
        lax.fori_loop(0, N_NODES // 16, _zeroA, 0)
        for t in range(2):
            plsc.store_scatter(At, [qv[pl.ds(t * 16, 16)]], atts[t])

        def _chunk(ci, _):
            base = pl.multiple_of(ci * CH, 8)
            pltpu.sync_copy(src_hbm.at[pl.ds(base, CH)], sbuf0)
            pltpu.sync_copy(dst_hbm.at[pl.ds(base, CH)], dbuf0)

            def _vec(k, __):
                sl = pl.ds(k * 16, 16)
                sv = sbuf0[sl]
                dv = dbuf0[sl]
                a_s = plsc.load_gather(At, [sv])
                a_d = plsc.load_gather(At, [dv])
                plsc.addupdate_scatter(Ct, [dv], a_s)
                plsc.addupdate_scatter(Ct, [sv], a_d)
                return 0
            lax.fori_loop(0, CH // 16, _vec, 0)
            return 0
        lax.fori_loop(0, N_EDGES // CH, _chunk, 0)

    bufs = ((sbuf0, dbuf0, ibuf0, sem0), (sbuf1, dbuf1, ibuf1, sem1))

    def _in_slices(ci):
        base = pl.multiple_of(c * EH + ci * CH, 8)
        return (src_hbm.at[pl.ds(base, CH)], dst_hbm.at[pl.ds(base, CH)],
                init_hbm.at[pl.ds(base, CH)])

    def _start_in(ci, bs):
        s0, s1, s2 = _in_slices(ci)
        pltpu.async_copy(s0, bs[0], bs[3])
        pltpu.async_copy(s1, bs[1], bs[3])
        pltpu.async_copy(s2, bs[2], bs[3])

    def _wait_in(ci, bs):
        s0, s1, s2 = _in_slices(ci)
        pltpu.make_async_copy(s0, bs[0], bs[3]).wait()
        pltpu.make_async_copy(s1, bs[1], bs[3]).wait()
        pltpu.make_async_copy(s2, bs[2], bs[3]).wait()

    def _compute(ci, bs):
        sb, db, ib = bs[0], bs[1], bs[2]

        def _vec(k, __):
            for u in range(2):
                sl = pl.ds((k * 2 + u) * 16, 16)
                cs = plsc.load_gather(Ct, [sb[sl]])
                cd = plsc.load_gather(Ct, [db[sl]])
                obuf[sl] = ib[sl] + cs + cd
            return 0
        lax.fori_loop(0, CH // 32, _vec, 0)
        gbase = pl.multiple_of(b * N_EDGES + c * EH + ci * CH, 8)
        pltpu.sync_copy(obuf, graph_hbm.at[pl.ds(gbase, CH)])

    _start_in(jnp.int32(0), bufs[0])

    def _pair(pi, _):
        ci0 = pi * 2
        _wait_in(ci0, bufs[0])
        _start_in(ci0 + 1, bufs[1])
        _compute(ci0, bufs[0])
        _wait_in(ci0 + 1, bufs[1])

        @pl.when(pi < NCHUNK // 2 - 1)
        def _():
            _start_in(ci0 + 2, bufs[0])
        _compute(ci0 + 1, bufs[1])
        return 0
    lax.fori_loop(0, NCHUNK // 2, _pair, 0)


_CB = 6400
_NB = N_EDGES // _CB


def _k4_body(graph_ref, T_ref, ngt_ref, need_ref):
    def count_ge(test):
        def blk(k, acc):
            v = graph_ref[:, pl.ds(k * _CB, _CB)]
            bv = lax.bitcast_convert_type(v, jnp.int32)
            return acc + jnp.sum((bv >= test).astype(jnp.int32), axis=1,
                                 keepdims=True)
        return lax.fori_loop(0, _NB, blk, jnp.zeros((B, 1), jnp.int32))

    cand = jnp.where(count_ge(jnp.full((B, 1), 1 << 30, jnp.int32)) >= K_TOP,
                     jnp.full((B, 1), 1 << 30, jnp.int32),
                     jnp.zeros((B, 1), jnp.int32))

    def radix_step(i, cand):
        shift = 28 - 2 * i
        t1 = cand | (1 << shift)
        t2 = cand | (2 << shift)
        t3 = cand | (3 << shift)

        def blk(k, accs):
            a1, a2, a3 = accs
            v = graph_ref[:, pl.ds(k * _CB, _CB)]
            bv = lax.bitcast_convert_type(v, jnp.int32)
            a1 = a1 + jnp.sum((bv >= t1).astype(jnp.int32), axis=1,
                              keepdims=True)
            a2 = a2 + jnp.sum((bv >= t2).astype(jnp.int32), axis=1,
                              keepdims=True)
            a3 = a3 + jnp.sum((bv >= t3).astype(jnp.int32), axis=1,
                              keepdims=True)
            return (a1, a2, a3)
        z = jnp.zeros((B, 1), jnp.int32)
        c1, c2, c3 = lax.fori_loop(0, _NB, blk, (z, z, z))
        return jnp.where(
            c3 >= K_TOP, t3,
            jnp.where(c2 >= K_TOP, t2, jnp.where(c1 >= K_TOP, t1, cand)))

    cand = lax.fori_loop(0, 15, radix_step, cand)
    ngt = count_ge(cand + 1)
    T_ref[...] = lax.bitcast_convert_type(cand, jnp.float32)
    ngt_ref[...] = ngt
    need_ref[...] = K_TOP - ngt


_k4_threshold = pl.pallas_call(
    _k4_body,
    out_shape=(
        jax.ShapeDtypeStruct((B, 1), jnp.float32),
        jax.ShapeDtypeStruct((B, 1), jnp.int32),
        jax.ShapeDtypeStruct((B, 1), jnp.int32),
    ),
)


_K5CH = 4000
_K5NCH = EH // _K5CH


@functools.partial(
    pl.kernel,
    out_type=(
        jax.ShapeDtypeStruct((B * 2 * CBUF,), jnp.float32),
        jax.ShapeDtypeStruct((B * 2 * CBUF,), jnp.int32),
        jax.ShapeDtypeStruct((B * 2 * CBUF,), jnp.float32),
        jax.ShapeDtypeStruct((B * 2 * CBUF,), jnp.int32),
        jax.ShapeDtypeStruct((B * 2 * 2 * 16,), jnp.int32),
    ),
    mesh=_mesh,
    compiler_params=_sc_params,
    scratch_types=[
        pltpu.VMEM((16,), jnp.float32),
        pltpu.VMEM((16,), jnp.int32),
        pltpu.VMEM((_K5CH,), jnp.float32),
        pltpu.VMEM((CBUF,), jnp.float32),
        pltpu.VMEM((CBUF,), jnp.int32),
        pltpu.VMEM((CBUF,), jnp.float32),
        pltpu.VMEM((CBUF,), jnp.int32),
        pltpu.VMEM((32,), jnp.int32),
    ],
)
def _k5_compact(graph_hbm, t_hbm, need_hbm,
                gtv_hbm, gti_hbm, eqv_hbm, eqi_hbm, ngt_hbm,
                Tv, needv, cbuf, gvb, gib, evb, eib, nsplat):
    b = lax.axis_index("s")
    c = lax.axis_index("c")
    pltpu.sync_copy(t_hbm, Tv)
    pltpu.sync_copy(need_hbm, needv)
    tb = plsc.load_gather(Tv, [_splat_i32(b)])
    nb = plsc.load_gather(needv, [_splat_i32(b)])

    def _zero(i, _):
        sl = pl.ds(i * 16, 16)
        gvb[sl] = jnp.zeros((16,), jnp.float32)
        gib[sl] = jnp.zeros((16,), jnp.int32)
        evb[sl] = jnp.zeros((16,), jnp.float32)
        eib[sl] = jnp.zeros((16,), jnp.int32)
        return 0
    lax.fori_loop(0, CBUF // 16, _zero, 0)

    iota16 = lax.broadcasted_iota(jnp.int32, (16,), 0)
    ebase = c * EH

    def _chunk(ci, carry):
        gb = pl.multiple_of(b * N_EDGES + ebase + ci * _K5CH, 8)
        pltpu.sync_copy(graph_hbm.at[pl.ds(gb, _K5CH)], cbuf)

        def _grp(k, cr):
            vs = [cbuf[pl.ds((k * 5 + i) * 16, 16)] for i in range(5)]
            pc = plsc.all_reduce_population_count(vs[0] >= tb)
            for i in range(1, 5):
                pc = pc + plsc.all_reduce_population_count(vs[i] >= tb)
            hits = pc[0]

            def _slow(cr2):
                for i in range(5):
                    v = vs[i]
                    ngt, mst, etot = cr2
                    m_gt = v > tb
                    m_eq = v == tb
                    gidx = (_splat_i32(ebase + ci * _K5CH + (k * 5 + i) * 16)
                            + iota16)
                    eqrank = plsc.cumsum(m_eq.astype(jnp.int32))
                    take = m_eq & ((_splat_i32(etot) + eqrank) <= nb)
                    plsc.store_compressed(gvb.at[pl.ds(ngt, 16)], v,
                                          mask=m_gt)
                    plsc.store_compressed(gib.at[pl.ds(ngt, 16)], gidx,
                                          mask=m_gt)
                    plsc.store_compressed(evb.at[pl.ds(mst, 16)], v,
                                          mask=take)
                    plsc.store_compressed(eib.at[pl.ds(mst, 16)], gidx,
                                          mask=take)
                    ngt = ngt + plsc.all_reduce_population_count(m_gt)[0]
                    mst = mst + plsc.all_reduce_population_count(take)[0]
                    etot = etot + plsc.all_reduce_population_count(m_eq)[0]
                    cr2 = (ngt, mst, etot)
                return cr2

            return lax.cond(hits > 0, _slow, lambda cr2: cr2, cr)
        return lax.fori_loop(0, _K5CH // 80, _grp, carry)

    ngt, mst, etot = lax.fori_loop(
        0, _K5NCH, _chunk, (jnp.int32(0), jnp.int32(0), jnp.int32(0)))
    nsplat[pl.ds(0, 16)] = _splat_i32(ngt)
    nsplat[pl.ds(16, 16)] = _splat_i32(etot)
    ob = pl.multiple_of((b * 2 + c) * CBUF, 8)
    pltpu.sync_copy(gvb, gtv_hbm.at[pl.ds(ob, CBUF)])
    pltpu.sync_copy(gib, gti_hbm.at[pl.ds(ob, CBUF)])
    pltpu.sync_copy(evb, eqv_hbm.at[pl.ds(ob, CBUF)])
    pltpu.sync_copy(eib, eqi_hbm.at[pl.ds(ob, CBUF)])
    pltpu.sync_copy(nsplat,
                    ngt_hbm.at[pl.ds(pl.multiple_of((b * 2 + c) * 32, 8), 32)])


def _k6_body(vrow_ref, vcol_ref, irow_ref, icol_ref, ngt_ref, delta_ref,
             w_ref, if_ref):
    vrow = vrow_ref[0]
    vcol = vcol_ref[0]
    irow = irow_ref[0]
    icol = icol_ref[0]
    cnts = ngt_ref[0]
    n0 = jnp.max(cnts[:, 0:16])
    e0 = jnp.max(cnts[:, 16:32])
    n1 = jnp.max(cnts[:, 32:48])
    e1 = jnp.max(cnts[:, 48:64])
    need = K_TOP - n0 - n1
    m0 = jnp.minimum(need, e0)
    m1 = need - e0

    one = jnp.int32(1)
    zero = jnp.int32(0)

    def _valid(pos):
        r0 = jnp.where(pos < n0, one, zero)
        r1 = jnp.where(pos - CBUF < n1, one, zero)
        r2 = jnp.where(pos - 2 * CBUF < m0, one, zero)
        r3 = jnp.where(pos - 3 * CBUF < m1, one, zero)
        lo = jnp.where(pos < CBUF, r0, r1)
        hi = jnp.where(pos < 3 * CBUF, r2, r3)
        return jnp.where(pos < 2 * CBUF, lo, hi)

    lane = lax.broadcasted_iota(jnp.int32, (1, NCAND), 1)
    valid_row = _valid(lane)
    sub = lax.broadcasted_iota(jnp.int32, (NCAND, 1), 0)
    valid_col = _valid(sub)

    gt_i = jnp.where(vrow > vcol, one, zero)
    eq_i = jnp.where(vrow == vcol, one, zero)
    lt_i = jnp.where(irow < icol, one, zero)
    beats = (gt_i + eq_i * lt_i) * valid_row
    pos = jnp.sum(beats, axis=1, keepdims=True)

    kk = lax.broadcasted_iota(jnp.int32, (1, K_TOP), 1)
    oh = jnp.where(pos == kk, one, zero) * valid_col
    ohf = oh.astype(jnp.float32)
    w = jnp.sum(vcol * ohf, axis=0, keepdims=True)
    idx = jnp.sum(icol * ohf, axis=0, keepdims=True)
    w_ref[0] = w + delta_ref[0, 0, 0]
    if_ref[0] = idx


_k6_sort = pl.pallas_call(
    _k6_body,
    grid=(B,),
    in_specs=[
        pl.BlockSpec((1, 1, NCAND), lambda b: (b, 0, 0)),
        pl.BlockSpec((1, NCAND, 1), lambda b: (b, 0, 0)),
        pl.BlockSpec((1, 1, NCAND), lambda b: (b, 0, 0)),
        pl.BlockSpec((1, NCAND, 1), lambda b: (b, 0, 0)),
        pl.BlockSpec((1, 1, 64), lambda b: (b, 0, 0)),
        pl.BlockSpec((1, 1, 1), lambda b: (0, 0, 0)),
    ],
    out_specs=(
        pl.BlockSpec((1, 1, K_TOP), lambda b: (b, 0, 0)),
        pl.BlockSpec((1, 1, K_TOP), lambda b: (b, 0, 0)),
    ),
    out_shape=(
        jax.ShapeDtypeStruct((B, 1, K_TOP), jnp.float32),
        jax.ShapeDtypeStruct((B, 1, K_TOP), jnp.float32),
    ),
)


_ROWS_PER_W = (B * K_TOP) // 32


@functools.partial(
    pl.kernel,
    out_type=jax.ShapeDtypeStruct((B * K_TOP, D_FEAT), jnp.float32),
    mesh=_mesh,
    compiler_params=_sc_params,
    scratch_types=[
        pltpu.VMEM((_ROWS_PER_W,), jnp.int32),
        pltpu.VMEM((_ROWS_PER_W,), jnp.int32),
        pltpu.VMEM((_ROWS_PER_W,), jnp.float32),
        pltpu.VMEM((_ROWS_PER_W, D_FEAT), jnp.float32),
        pltpu.SemaphoreType.DMA,
    ],
)
def _k7_gather(topidx_hbm, topw_hbm, src_hbm, emb_hbm, out_hbm,
               ev, nv, wv, rows, sem):
    b = lax.axis_index("s")
    c = lax.axis_index("c")
    w = b * 2 + c
    base = pl.multiple_of(w * _ROWS_PER_W, 8)
    pltpu.sync_copy(topidx_hbm.at[pl.ds(base, _ROWS_PER_W)], ev)
    pltpu.sync_copy(topw_hbm.at[pl.ds(base, _ROWS_PER_W)], wv)
    pltpu.async_copy(src_hbm.at[ev], nv, sem).wait()
    pltpu.async_copy(emb_hbm.at[nv], rows, sem).wait()

    def _scale_row(i, _):
        ws = plsc.load_gather(wv, [_splat_i32(i)])
        for ch in range(D_FEAT // 16):
            sl = pl.ds(ch * 16, 16)
            rows[i, sl] = rows[i, sl] * ws
        return 0
    lax.fori_loop(0, _ROWS_PER_W, _scale_row, 0)
    pltpu.sync_copy(rows, out_hbm.at[pl.ds(base, _ROWS_PER_W)])


def kernel(list_questions, attention_question, edge_index, num_max_nodes,
           init_graph_tensor, node_embedding, w_importance):
    f32 = jnp.float32
    i32 = jnp.int32
    src = edge_index[0].astype(i32)
    dst = edge_index[1].astype(i32)

    qpad = jnp.zeros((B, QP), i32).at[:, :Q].set(
        list_questions.astype(i32)).reshape(B * QP)
    apad = jnp.zeros((B, QP), f32).at[:, :Q].set(
        attention_question).reshape(B * QP)
    wpad = jnp.zeros((QP,), f32).at[:Q].set(w_importance)

    graph = _k23_graph(qpad, apad, wpad, src, dst, init_graph_tensor)
    t_b, ngt_b, need_b = _k4_threshold(graph.reshape(B, N_EDGES))
    gtv, gti, eqv, eqi, ngt = _k5_compact(
        graph, t_b.reshape(B), need_b.reshape(B))
    gtv = gtv.reshape(B, 2 * CBUF)
    gti = gti.reshape(B, 2 * CBUF)
    eqv = eqv.reshape(B, 2 * CBUF)
    eqi = eqi.reshape(B, 2 * CBUF)
    ngt = ngt.reshape(B, 64)

    cat_v = jnp.concatenate([gtv, eqv], axis=1)
    cat_i = jnp.concatenate([gti, eqi], axis=1).astype(f32)
    delta = jnp.asarray(num_max_nodes - K_TOP, f32).reshape(1, 1, 1)
    top_w, top_if = _k6_sort(
        cat_v[:, None, :], cat_v[:, :, None],
        cat_i[:, None, :], cat_i[:, :, None],
        ngt[:, None, :], delta)

    top_idx = top_if.reshape(B * K_TOP).astype(i32)
    out = _k7_gather(top_idx, top_w.reshape(B * K_TOP), src, node_embedding)
    return out.reshape(B, K_TOP, D_FEAT)


# speedup vs baseline: 1.4526x; 1.0431x over previous
import functools

import jax
import jax.numpy as jnp
from jax import lax
from jax.experimental import pallas as pl
from jax.experimental.pallas import tpu as pltpu
from jax.experimental.pallas import tpu_sc as plsc

N_NODES = 10000
N_EDGES = 160000
B = 16
Q = 20
K_TOP = 256
D_FEAT = 256
PROP_THRESHOLD = 0.5
ATTEN_COEF = 0.25

QP = 32
EH = N_EDGES // 2
CH = 4000
NCHUNK = EH // CH
CBUF = 272
NCAND = 4 * CBUF

_mesh = plsc.VectorSubcoreMesh(core_axis_name="c", subcore_axis_name="s")
_sc_params = pltpu.CompilerParams(needs_layout_passes=False)


def _splat_i32(x):
    return jnp.zeros((16,), jnp.int32) + x


def _splat_f32(x):
    return jnp.zeros((16,), jnp.float32) + x


@functools.partial(
    pl.kernel,
    out_type=jax.ShapeDtypeStruct((B * N_EDGES,), jnp.float32),
    mesh=_mesh,
    compiler_params=_sc_params,
    scratch_types=[
        pltpu.VMEM((QP,), jnp.int32),
        pltpu.VMEM((QP,), jnp.float32),
        pltpu.VMEM((QP,), jnp.float32),
        pltpu.VMEM((QP,), jnp.float32),
        pltpu.VMEM((N_NODES,), jnp.float32),
        pltpu.VMEM((N_NODES,), jnp.float32),
        pltpu.VMEM((CH,), jnp.int32),
        pltpu.VMEM((CH,), jnp.int32),
        pltpu.VMEM((CH,), jnp.float32),
        pltpu.VMEM((CH,), jnp.int32),
        pltpu.VMEM((CH,), jnp.int32),
        pltpu.VMEM((CH,), jnp.float32),
        pltpu.VMEM((CH,), jnp.float32),
        pltpu.SemaphoreType.DMA,
        pltpu.SemaphoreType.DMA,
    ],
)
def _k23_graph(q_hbm, a_hbm, w_hbm, src_hbm, dst_hbm, init_hbm, graph_hbm,
               qv, av, wv, gv, Ct, At, sbuf0, dbuf0, ibuf0,
               sbuf1, dbuf1, ibuf1, obuf, sem0, sem1):
    b = lax.axis_index("s")
    c = lax.axis_index("c")
    pltpu.sync_copy(q_hbm.at[pl.ds(pl.multiple_of(b * QP, 8), QP)], qv)
    pltpu.sync_copy(a_hbm.at[pl.ds(pl.multiple_of(b * QP, 8), QP)], av)
    pltpu.sync_copy(w_hbm, wv)

    for t in range(2):
        sl = pl.ds(t * 16, 16)
        x = av[sl] * wv[sl]
        imp = 1.0 / (1.0 + jnp.exp(-x))
        g = jnp.where(imp >= PROP_THRESHOLD, imp, 0.0)
        if t == 1:
            lane = lax.broadcasted_iota(jnp.int32, (16,), 0)
            g = jnp.where(lane < (Q - 16), g, 0.0)
        gv[sl] = g

    def _zero(i, _):
        Ct[pl.ds(i * 16, 16)] = jnp.zeros((16,), jnp.float32)
        return 0
    lax.fori_loop(0, N_NODES // 16, _zero, 0)

    for t in range(2):
        sl = pl.ds(t * 16, 16)
        plsc.addupdate_scatter(Ct, [qv[sl]], gv[sl])

    atts = []
    n_att = jnp.int32(0)
    for t in range(2):
        sl = pl.ds(t * 16, 16)
        qch = qv[sl]
        v = jnp.zeros((16,), jnp.float32)
        for j in range(Q):
            qs = plsc.load_gather(qv, [jnp.full((16,), j, jnp.int32)])
            gs = plsc.load_gather(gv, [jnp.full((16,), j, jnp.int32)])
            v = v + jnp.where(qch == qs, gs, 0.0)
        a = ATTEN_COEF * v
        a = jnp.where(a >= PROP_THRESHOLD, a, 0.0)
        atts.append(a)
        n_att = n_att + jnp.max(plsc.all_reduce_population_count(a > 0.0))

    @pl.when(n_att > 0)
    def _slow_hop():
        def _zeroA(i, _):
            At[pl.ds(i * 16, 16)] = jnp.zeros((16,), jnp.float32)
            return 0
        lax.fori_loop(0, N_NODES // 16, _zeroA, 0)
        for t in range(2):
            plsc.store_scatter(At, [qv[pl.ds(t * 16, 16)]], atts[t])

        def _chunk(ci, _):
            base = pl.multiple_of(ci * CH, 8)
            pltpu.sync_copy(src_hbm.at[pl.ds(base, CH)], sbuf0)
            pltpu.sync_copy(dst_hbm.at[pl.ds(base, CH)], dbuf0)

            def _vec(k, __):
                sl = pl.ds(k * 16, 16)
                sv = sbuf0[sl]
                dv = dbuf0[sl]
                a_s = plsc.load_gather(At, [sv])
                a_d = plsc.load_gather(At, [dv])
                plsc.addupdate_scatter(Ct, [dv], a_s)
                plsc.addupdate_scatter(Ct, [sv], a_d)
                return 0
            lax.fori_loop(0, CH // 16, _vec, 0)
            return 0
        lax.fori_loop(0, N_EDGES // CH, _chunk, 0)

    bufs = ((sbuf0, dbuf0, ibuf0, sem0), (sbuf1, dbuf1, ibuf1, sem1))

    def _in_slices(ci):
        base = pl.multiple_of(c * EH + ci * CH, 8)
        return (src_hbm.at[pl.ds(base, CH)], dst_hbm.at[pl.ds(base, CH)],
                init_hbm.at[pl.ds(base, CH)])

    def _start_in(ci, bs):
        s0, s1, s2 = _in_slices(ci)
        pltpu.async_copy(s0, bs[0], bs[3])
        pltpu.async_copy(s1, bs[1], bs[3])
        pltpu.async_copy(s2, bs[2], bs[3])

    def _wait_in(ci, bs):
        s0, s1, s2 = _in_slices(ci)
        pltpu.make_async_copy(s0, bs[0], bs[3]).wait()
        pltpu.make_async_copy(s1, bs[1], bs[3]).wait()
        pltpu.make_async_copy(s2, bs[2], bs[3]).wait()

    def _compute(ci, bs):
        sb, db, ib = bs[0], bs[1], bs[2]

        def _vec(k, __):
            for u in range(2):
                sl = pl.ds((k * 2 + u) * 16, 16)
                cs = plsc.load_gather(Ct, [sb[sl]])
                cd = plsc.load_gather(Ct, [db[sl]])
                obuf[sl] = ib[sl] + cs + cd
            return 0
        lax.fori_loop(0, CH // 32, _vec, 0)
        gbase = pl.multiple_of(b * N_EDGES + c * EH + ci * CH, 8)
        pltpu.sync_copy(obuf, graph_hbm.at[pl.ds(gbase, CH)])

    _start_in(jnp.int32(0), bufs[0])

    def _pair(pi, _):
        ci0 = pi * 2
        _wait_in(ci0, bufs[0])
        _start_in(ci0 + 1, bufs[1])
        _compute(ci0, bufs[0])
        _wait_in(ci0 + 1, bufs[1])

        @pl.when(pi < NCHUNK // 2 - 1)
        def _():
            _start_in(ci0 + 2, bufs[0])
        _compute(ci0 + 1, bufs[1])
        return 0
    lax.fori_loop(0, NCHUNK // 2, _pair, 0)


_CB = 16000
_NB = N_EDGES // _CB


def _k4_body(graph_ref, T_ref, ngt_ref, need_ref):
    def count_ge(test):
        def blk(k, acc):
            v = graph_ref[:, pl.ds(k * _CB, _CB)]
            bv = lax.bitcast_convert_type(v, jnp.int32)
            return acc + jnp.sum((bv >= test).astype(jnp.int32), axis=1,
                                 keepdims=True)
        return lax.fori_loop(0, _NB, blk, jnp.zeros((B, 1), jnp.int32))

    cand = jnp.where(count_ge(jnp.full((B, 1), 1 << 30, jnp.int32)) >= K_TOP,
                     jnp.full((B, 1), 1 << 30, jnp.int32),
                     jnp.zeros((B, 1), jnp.int32))

    def radix_step(i, cand):
        shift = 28 - 2 * i
        t1 = cand | (1 << shift)
        t2 = cand | (2 << shift)
        t3 = cand | (3 << shift)

        def blk(k, accs):
            a1, a2, a3 = accs
            v = graph_ref[:, pl.ds(k * _CB, _CB)]
            bv = lax.bitcast_convert_type(v, jnp.int32)
            a1 = a1 + jnp.sum((bv >= t1).astype(jnp.int32), axis=1,
                              keepdims=True)
            a2 = a2 + jnp.sum((bv >= t2).astype(jnp.int32), axis=1,
                              keepdims=True)
            a3 = a3 + jnp.sum((bv >= t3).astype(jnp.int32), axis=1,
                              keepdims=True)
            return (a1, a2, a3)
        z = jnp.zeros((B, 1), jnp.int32)
        c1, c2, c3 = lax.fori_loop(0, _NB, blk, (z, z, z))
        return jnp.where(
            c3 >= K_TOP, t3,
            jnp.where(c2 >= K_TOP, t2, jnp.where(c1 >= K_TOP, t1, cand)))

    cand = lax.fori_loop(0, 15, radix_step, cand)
    ngt = count_ge(cand + 1)
    T_ref[...] = lax.bitcast_convert_type(cand, jnp.float32)
    ngt_ref[...] = ngt
    need_ref[...] = K_TOP - ngt


_k4_threshold = pl.pallas_call(
    _k4_body,
    out_shape=(
        jax.ShapeDtypeStruct((B, 1), jnp.float32),
        jax.ShapeDtypeStruct((B, 1), jnp.int32),
        jax.ShapeDtypeStruct((B, 1), jnp.int32),
    ),
)


_K5CH = 4000
_K5NCH = EH // _K5CH


@functools.partial(
    pl.kernel,
    out_type=(
        jax.ShapeDtypeStruct((B * 2 * CBUF,), jnp.float32),
        jax.ShapeDtypeStruct((B * 2 * CBUF,), jnp.int32),
        jax.ShapeDtypeStruct((B * 2 * CBUF,), jnp.float32),
        jax.ShapeDtypeStruct((B * 2 * CBUF,), jnp.int32),
        jax.ShapeDtypeStruct((B * 2 * 2 * 16,), jnp.int32),
    ),
    mesh=_mesh,
    compiler_params=_sc_params,
    scratch_types=[
        pltpu.VMEM((16,), jnp.float32),
        pltpu.VMEM((16,), jnp.int32),
        pltpu.VMEM((_K5CH,), jnp.float32),
        pltpu.VMEM((CBUF,), jnp.float32),
        pltpu.VMEM((CBUF,), jnp.int32),
        pltpu.VMEM((CBUF,), jnp.float32),
        pltpu.VMEM((CBUF,), jnp.int32),
        pltpu.VMEM((32,), jnp.int32),
    ],
)
def _k5_compact(graph_hbm, t_hbm, need_hbm,
                gtv_hbm, gti_hbm, eqv_hbm, eqi_hbm, ngt_hbm,
                Tv, needv, cbuf, gvb, gib, evb, eib, nsplat):
    b = lax.axis_index("s")
    c = lax.axis_index("c")
    pltpu.sync_copy(t_hbm, Tv)
    pltpu.sync_copy(need_hbm, needv)
    tb = plsc.load_gather(Tv, [_splat_i32(b)])
    nb = plsc.load_gather(needv, [_splat_i32(b)])

    def _zero(i, _):
        sl = pl.ds(i * 16, 16)
        gvb[sl] = jnp.zeros((16,), jnp.float32)
        gib[sl] = jnp.zeros((16,), jnp.int32)
        evb[sl] = jnp.zeros((16,), jnp.float32)
        eib[sl] = jnp.zeros((16,), jnp.int32)
        return 0
    lax.fori_loop(0, CBUF // 16, _zero, 0)

    iota16 = lax.broadcasted_iota(jnp.int32, (16,), 0)
    ebase = c * EH

    def _chunk(ci, carry):
        gb = pl.multiple_of(b * N_EDGES + ebase + ci * _K5CH, 8)
        pltpu.sync_copy(graph_hbm.at[pl.ds(gb, _K5CH)], cbuf)

        def _grp(k, cr):
            vs = [cbuf[pl.ds((k * 5 + i) * 16, 16)] for i in range(5)]
            pc = plsc.all_reduce_population_count(vs[0] >= tb)
            for i in range(1, 5):
                pc = pc + plsc.all_reduce_population_count(vs[i] >= tb)
            hits = pc[0]

            def _slow(cr2):
                for i in range(5):
                    v = vs[i]
                    ngt, mst, etot = cr2
                    m_gt = v > tb
                    m_eq = v == tb
                    gidx = (_splat_i32(ebase + ci * _K5CH + (k * 5 + i) * 16)
                            + iota16)
                    eqrank = plsc.cumsum(m_eq.astype(jnp.int32))
                    take = m_eq & ((_splat_i32(etot) + eqrank) <= nb)
                    plsc.store_compressed(gvb.at[pl.ds(ngt, 16)], v,
                                          mask=m_gt)
                    plsc.store_compressed(gib.at[pl.ds(ngt, 16)], gidx,
                                          mask=m_gt)
                    plsc.store_compressed(evb.at[pl.ds(mst, 16)], v,
                                          mask=take)
                    plsc.store_compressed(eib.at[pl.ds(mst, 16)], gidx,
                                          mask=take)
                    ngt = ngt + plsc.all_reduce_population_count(m_gt)[0]
                    mst = mst + plsc.all_reduce_population_count(take)[0]
                    etot = etot + plsc.all_reduce_population_count(m_eq)[0]
                    cr2 = (ngt, mst, etot)
                return cr2

            return lax.cond(hits > 0, _slow, lambda cr2: cr2, cr)
        return lax.fori_loop(0, _K5CH // 80, _grp, carry)

    ngt, mst, etot = lax.fori_loop(
        0, _K5NCH, _chunk, (jnp.int32(0), jnp.int32(0), jnp.int32(0)))
    nsplat[pl.ds(0, 16)] = _splat_i32(ngt)
    nsplat[pl.ds(16, 16)] = _splat_i32(etot)
    ob = pl.multiple_of((b * 2 + c) * CBUF, 8)
    pltpu.sync_copy(gvb, gtv_hbm.at[pl.ds(ob, CBUF)])
    pltpu.sync_copy(gib, gti_hbm.at[pl.ds(ob, CBUF)])
    pltpu.sync_copy(evb, eqv_hbm.at[pl.ds(ob, CBUF)])
    pltpu.sync_copy(eib, eqi_hbm.at[pl.ds(ob, CBUF)])
    pltpu.sync_copy(nsplat,
                    ngt_hbm.at[pl.ds(pl.multiple_of((b * 2 + c) * 32, 8), 32)])


def _k6_body(vrow_ref, vcol_ref, irow_ref, icol_ref, ngt_ref, delta_ref,
             w_ref, if_ref):
    vrow = vrow_ref[0]
    vcol = vcol_ref[0]
    irow = irow_ref[0]
    icol = icol_ref[0]
    cnts = ngt_ref[0]
    n0 = jnp.max(cnts[:, 0:16])
    e0 = jnp.max(cnts[:, 16:32])
    n1 = jnp.max(cnts[:, 32:48])
    e1 = jnp.max(cnts[:, 48:64])
    need = K_TOP - n0 - n1
    m0 = jnp.minimum(need, e0)
    m1 = need - e0

    one = jnp.int32(1)
    zero = jnp.int32(0)

    def _valid(pos):
        r0 = jnp.where(pos < n0, one, zero)
        r1 = jnp.where(pos - CBUF < n1, one, zero)
        r2 = jnp.where(pos - 2 * CBUF < m0, one, zero)
        r3 = jnp.where(pos - 3 * CBUF < m1, one, zero)
        lo = jnp.where(pos < CBUF, r0, r1)
        hi = jnp.where(pos < 3 * CBUF, r2, r3)
        return jnp.where(pos < 2 * CBUF, lo, hi)

    lane = lax.broadcasted_iota(jnp.int32, (1, NCAND), 1)
    valid_row = _valid(lane)
    sub = lax.broadcasted_iota(jnp.int32, (NCAND, 1), 0)
    valid_col = _valid(sub)

    gt_i = jnp.where(vrow > vcol, one, zero)
    eq_i = jnp.where(vrow == vcol, one, zero)
    lt_i = jnp.where(irow < icol, one, zero)
    beats = (gt_i + eq_i * lt_i) * valid_row
    pos = jnp.sum(beats, axis=1, keepdims=True)

    kk = lax.broadcasted_iota(jnp.int32, (1, K_TOP), 1)
    oh = jnp.where(pos == kk, one, zero) * valid_col
    ohf = oh.astype(jnp.float32)
    w = jnp.sum(vcol * ohf, axis=0, keepdims=True)
    idx = jnp.sum(icol * ohf, axis=0, keepdims=True)
    w_ref[0] = w + delta_ref[0, 0, 0]
    if_ref[0] = idx


_k6_sort = pl.pallas_call(
    _k6_body,
    grid=(B,),
    in_specs=[
        pl.BlockSpec((1, 1, NCAND), lambda b: (b, 0, 0)),
        pl.BlockSpec((1, NCAND, 1), lambda b: (b, 0, 0)),
        pl.BlockSpec((1, 1, NCAND), lambda b: (b, 0, 0)),
        pl.BlockSpec((1, NCAND, 1), lambda b: (b, 0, 0)),
        pl.BlockSpec((1, 1, 64), lambda b: (b, 0, 0)),
        pl.BlockSpec((1, 1, 1), lambda b: (0, 0, 0)),
    ],
    out_specs=(
        pl.BlockSpec((1, 1, K_TOP), lambda b: (b, 0, 0)),
        pl.BlockSpec((1, 1, K_TOP), lambda b: (b, 0, 0)),
    ),
    out_shape=(
        jax.ShapeDtypeStruct((B, 1, K_TOP), jnp.float32),
        jax.ShapeDtypeStruct((B, 1, K_TOP), jnp.float32),
    ),
)


_ROWS_PER_W = (B * K_TOP) // 32


@functools.partial(
    pl.kernel,
    out_type=jax.ShapeDtypeStruct((B * K_TOP, D_FEAT), jnp.float32),
    mesh=_mesh,
    compiler_params=_sc_params,
    scratch_types=[
        pltpu.VMEM((_ROWS_PER_W,), jnp.int32),
        pltpu.VMEM((_ROWS_PER_W,), jnp.int32),
        pltpu.VMEM((_ROWS_PER_W,), jnp.float32),
        pltpu.VMEM((_ROWS_PER_W, D_FEAT), jnp.float32),
        pltpu.SemaphoreType.DMA,
    ],
)
def _k7_gather(topidx_hbm, topw_hbm, src_hbm, emb_hbm, out_hbm,
               ev, nv, wv, rows, sem):
    b = lax.axis_index("s")
    c = lax.axis_index("c")
    w = b * 2 + c
    base = pl.multiple_of(w * _ROWS_PER_W, 8)
    pltpu.sync_copy(topidx_hbm.at[pl.ds(base, _ROWS_PER_W)], ev)
    pltpu.sync_copy(topw_hbm.at[pl.ds(base, _ROWS_PER_W)], wv)
    pltpu.async_copy(src_hbm.at[ev], nv, sem).wait()
    pltpu.async_copy(emb_hbm.at[nv], rows, sem).wait()

    def _scale_row(i, _):
        ws = plsc.load_gather(wv, [_splat_i32(i)])
        for ch in range(D_FEAT // 16):
            sl = pl.ds(ch * 16, 16)
            rows[i, sl] = rows[i, sl] * ws
        return 0
    lax.fori_loop(0, _ROWS_PER_W, _scale_row, 0)
    pltpu.sync_copy(rows, out_hbm.at[pl.ds(base, _ROWS_PER_W)])


def kernel(list_questions, attention_question, edge_index, num_max_nodes,
           init_graph_tensor, node_embedding, w_importance):
    f32 = jnp.float32
    i32 = jnp.int32
    src = edge_index[0].astype(i32)
    dst = edge_index[1].astype(i32)

    qpad = jnp.zeros((B, QP), i32).at[:, :Q].set(
        list_questions.astype(i32)).reshape(B * QP)
    apad = jnp.zeros((B, QP), f32).at[:, :Q].set(
        attention_question).reshape(B * QP)
    wpad = jnp.zeros((QP,), f32).at[:Q].set(w_importance)

    graph = _k23_graph(qpad, apad, wpad, src, dst, init_graph_tensor)
    t_b, ngt_b, need_b = _k4_threshold(graph.reshape(B, N_EDGES))
    gtv, gti, eqv, eqi, ngt = _k5_compact(
        graph, t_b.reshape(B), need_b.reshape(B))
    gtv = gtv.reshape(B, 2 * CBUF)
    gti = gti.reshape(B, 2 * CBUF)
    eqv = eqv.reshape(B, 2 * CBUF)
    eqi = eqi.reshape(B, 2 * CBUF)
    ngt = ngt.reshape(B, 64)

    cat_v = jnp.concatenate([gtv, eqv], axis=1)
    cat_i = jnp.concatenate([gti, eqi], axis=1).astype(f32)
    delta = jnp.asarray(num_max_nodes - K_TOP, f32).reshape(1, 1, 1)
    top_w, top_if = _k6_sort(
        cat_v[:, None, :], cat_v[:, :, None],
        cat_i[:, None, :], cat_i[:, :, None],
        ngt[:, None, :], delta)

    top_idx = top_if.reshape(B * K_TOP).astype(i32)
    out = _k7_gather(top_idx, top_w.reshape(B * K_TOP), src, node_embedding)
    return out.reshape(B, K_TOP, D_FEAT)


# speedup vs baseline: 1.5108x; 1.0401x over previous
import functools

import jax
import jax.numpy as jnp
from jax import lax
from jax.experimental import pallas as pl
from jax.experimental.pallas import tpu as pltpu
from jax.experimental.pallas import tpu_sc as plsc

N_NODES = 10000
N_EDGES = 160000
B = 16
Q = 20
K_TOP = 256
D_FEAT = 256
PROP_THRESHOLD = 0.5
ATTEN_COEF = 0.25

QP = 32
EH = N_EDGES // 2
CH = 4000
NCHUNK = EH // CH
CBUF = 272
NCAND = 4 * CBUF

_mesh = plsc.VectorSubcoreMesh(core_axis_name="c", subcore_axis_name="s")
_sc_params = pltpu.CompilerParams(needs_layout_passes=False)


def _splat_i32(x):
    return jnp.zeros((16,), jnp.int32) + x


def _splat_f32(x):
    return jnp.zeros((16,), jnp.float32) + x


@functools.partial(
    pl.kernel,
    out_type=jax.ShapeDtypeStruct((B * N_EDGES,), jnp.float32),
    mesh=_mesh,
    compiler_params=_sc_params,
    scratch_types=[
        pltpu.VMEM((QP,), jnp.int32),
        pltpu.VMEM((QP,), jnp.float32),
        pltpu.VMEM((QP,), jnp.float32),
        pltpu.VMEM((QP,), jnp.float32),
        pltpu.VMEM((N_NODES,), jnp.float32),
        pltpu.VMEM((N_NODES,), jnp.float32),
        pltpu.VMEM((CH,), jnp.int32),
        pltpu.VMEM((CH,), jnp.int32),
        pltpu.VMEM((CH,), jnp.float32),
        pltpu.VMEM((CH,), jnp.int32),
        pltpu.VMEM((CH,), jnp.int32),
        pltpu.VMEM((CH,), jnp.float32),
        pltpu.VMEM((CH,), jnp.float32),
        pltpu.SemaphoreType.DMA,
        pltpu.SemaphoreType.DMA,
    ],
)
def _k23_graph(q_hbm, a_hbm, w_hbm, src_hbm, dst_hbm, init_hbm, graph_hbm,
               qv, av, wv, gv, Ct, At, sbuf0, dbuf0, ibuf0,
               sbuf1, dbuf1, ibuf1, obuf, sem0, sem1):
    b = lax.axis_index("s")
    c = lax.axis_index("c")
    pltpu.sync_copy(q_hbm.at[pl.ds(pl.multiple_of(b * QP, 8), QP)], qv)
    pltpu.sync_copy(a_hbm.at[pl.ds(pl.multiple_of(b * QP, 8), QP)], av)
    pltpu.sync_copy(w_hbm, wv)

    for t in range(2):
        sl = pl.ds(t * 16, 16)
        x = av[sl] * wv[sl]
        imp = 1.0 / (1.0 + jnp.exp(-x))
        g = jnp.where(imp >= PROP_THRESHOLD, imp, 0.0)
        if t == 1:
            lane = lax.broadcasted_iota(jnp.int32, (16,), 0)
            g = jnp.where(lane < (Q - 16), g, 0.0)
        gv[sl] = g

    def _zero(i, _):
        Ct[pl.ds(i * 16, 16)] = jnp.zeros((16,), jnp.float32)
        return 0
    lax.fori_loop(0, N_NODES // 16, _zero, 0)

    for t in range(2):
        sl = pl.ds(t * 16, 16)
        plsc.addupdate_scatter(Ct, [qv[sl]], gv[sl])

    atts = []
    n_att = jnp.int32(0)
    for t in range(2):
        sl = pl.ds(t * 16, 16)
        qch = qv[sl]
        v = jnp.zeros((16,), jnp.float32)
        for j in range(Q):
            qs = plsc.load_gather(qv, [jnp.full((16,), j, jnp.int32)])
            gs = plsc.load_gather(gv, [jnp.full((16,), j, jnp.int32)])
            v = v + jnp.where(qch == qs, gs, 0.0)
        a = ATTEN_COEF * v
        a = jnp.where(a >= PROP_THRESHOLD, a, 0.0)
        atts.append(a)
        n_att = n_att + jnp.max(plsc.all_reduce_population_count(a > 0.0))

    @pl.when(n_att > 0)
    def _slow_hop():
        def _zeroA(i, _):
            At[pl.ds(i * 16, 16)] = jnp.zeros((16,), jnp.float32)
            return 0
        lax.fori_loop(0, N_NODES // 16, _zeroA, 0)
        for t in range(2):
            plsc.store_scatter(At, [qv[pl.ds(t * 16, 16)]], atts[t])

        def _chunk(ci, _):
            base = pl.multiple_of(ci * CH, 8)
            pltpu.sync_copy(src_hbm.at[pl.ds(base, CH)], sbuf0)
            pltpu.sync_copy(dst_hbm.at[pl.ds(base, CH)], dbuf0)

            def _vec(k, __):
                sl = pl.ds(k * 16, 16)
                sv = sbuf0[sl]
                dv = dbuf0[sl]
                a_s = plsc.load_gather(At, [sv])
                a_d = plsc.load_gather(At, [dv])
                plsc.addupdate_scatter(Ct, [dv], a_s)
                plsc.addupdate_scatter(Ct, [sv], a_d)
                return 0
            lax.fori_loop(0, CH // 16, _vec, 0)
            return 0
        lax.fori_loop(0, N_EDGES // CH, _chunk, 0)

    bufs = ((sbuf0, dbuf0, ibuf0, sem0), (sbuf1, dbuf1, ibuf1, sem1))

    def _in_slices(ci):
        base = pl.multiple_of(c * EH + ci * CH, 8)
        return (src_hbm.at[pl.ds(base, CH)], dst_hbm.at[pl.ds(base, CH)],
                init_hbm.at[pl.ds(base, CH)])

    def _start_in(ci, bs):
        s0, s1, s2 = _in_slices(ci)
        pltpu.async_copy(s0, bs[0], bs[3])
        pltpu.async_copy(s1, bs[1], bs[3])
        pltpu.async_copy(s2, bs[2], bs[3])

    def _wait_in(ci, bs):
        s0, s1, s2 = _in_slices(ci)
        pltpu.make_async_copy(s0, bs[0], bs[3]).wait()
        pltpu.make_async_copy(s1, bs[1], bs[3]).wait()
        pltpu.make_async_copy(s2, bs[2], bs[3]).wait()

    def _compute(ci, bs):
        sb, db, ib = bs[0], bs[1], bs[2]

        def _vec(k, __):
            for u in range(2):
                sl = pl.ds((k * 2 + u) * 16, 16)
                cs = plsc.load_gather(Ct, [sb[sl]])
                cd = plsc.load_gather(Ct, [db[sl]])
                obuf[sl] = ib[sl] + cs + cd
            return 0
        lax.fori_loop(0, CH // 32, _vec, 0)
        gbase = pl.multiple_of(b * N_EDGES + c * EH + ci * CH, 8)
        pltpu.sync_copy(obuf, graph_hbm.at[pl.ds(gbase, CH)])

    _start_in(jnp.int32(0), bufs[0])

    def _pair(pi, _):
        ci0 = pi * 2
        _wait_in(ci0, bufs[0])
        _start_in(ci0 + 1, bufs[1])
        _compute(ci0, bufs[0])
        _wait_in(ci0 + 1, bufs[1])

        @pl.when(pi < NCHUNK // 2 - 1)
        def _():
            _start_in(ci0 + 2, bufs[0])
        _compute(ci0 + 1, bufs[1])
        return 0
    lax.fori_loop(0, NCHUNK // 2, _pair, 0)


_CB = 32000
_NB = N_EDGES // _CB


def _k4_body(graph_ref, T_ref, ngt_ref, need_ref):
    def count_ge(test):
        def blk(k, acc):
            v = graph_ref[:, pl.ds(k * _CB, _CB)]
            bv = lax.bitcast_convert_type(v, jnp.int32)
            return acc + jnp.sum((bv >= test).astype(jnp.int32), axis=1,
                                 keepdims=True)
        return lax.fori_loop(0, _NB, blk, jnp.zeros((B, 1), jnp.int32))

    cand = jnp.where(count_ge(jnp.full((B, 1), 1 << 30, jnp.int32)) >= K_TOP,
                     jnp.full((B, 1), 1 << 30, jnp.int32),
                     jnp.zeros((B, 1), jnp.int32))

    def radix_step(i, cand):
        shift = 28 - 2 * i
        t1 = cand | (1 << shift)
        t2 = cand | (2 << shift)
        t3 = cand | (3 << shift)

        def blk(k, accs):
            a1, a2, a3 = accs
            v = graph_ref[:, pl.ds(k * _CB, _CB)]
            bv = lax.bitcast_convert_type(v, jnp.int32)
            a1 = a1 + jnp.sum((bv >= t1).astype(jnp.int32), axis=1,
                              keepdims=True)
            a2 = a2 + jnp.sum((bv >= t2).astype(jnp.int32), axis=1,
                              keepdims=True)
            a3 = a3 + jnp.sum((bv >= t3).astype(jnp.int32), axis=1,
                              keepdims=True)
            return (a1, a2, a3)
        z = jnp.zeros((B, 1), jnp.int32)
        c1, c2, c3 = lax.fori_loop(0, _NB, blk, (z, z, z))
        return jnp.where(
            c3 >= K_TOP, t3,
            jnp.where(c2 >= K_TOP, t2, jnp.where(c1 >= K_TOP, t1, cand)))

    cand = lax.fori_loop(0, 15, radix_step, cand)
    ngt = count_ge(cand + 1)
    T_ref[...] = lax.bitcast_convert_type(cand, jnp.float32)
    ngt_ref[...] = ngt
    need_ref[...] = K_TOP - ngt


_k4_threshold = pl.pallas_call(
    _k4_body,
    out_shape=(
        jax.ShapeDtypeStruct((B, 1), jnp.float32),
        jax.ShapeDtypeStruct((B, 1), jnp.int32),
        jax.ShapeDtypeStruct((B, 1), jnp.int32),
    ),
)


_K5CH = 4000
_K5NCH = EH // _K5CH


@functools.partial(
    pl.kernel,
    out_type=(
        jax.ShapeDtypeStruct((B * 2 * CBUF,), jnp.float32),
        jax.ShapeDtypeStruct((B * 2 * CBUF,), jnp.int32),
        jax.ShapeDtypeStruct((B * 2 * CBUF,), jnp.float32),
        jax.ShapeDtypeStruct((B * 2 * CBUF,), jnp.int32),
        jax.ShapeDtypeStruct((B * 2 * 2 * 16,), jnp.int32),
    ),
    mesh=_mesh,
    compiler_params=_sc_params,
    scratch_types=[
        pltpu.VMEM((16,), jnp.float32),
        pltpu.VMEM((16,), jnp.int32),
        pltpu.VMEM((_K5CH,), jnp.float32),
        pltpu.VMEM((CBUF,), jnp.float32),
        pltpu.VMEM((CBUF,), jnp.int32),
        pltpu.VMEM((CBUF,), jnp.float32),
        pltpu.VMEM((CBUF,), jnp.int32),
        pltpu.VMEM((32,), jnp.int32),
    ],
)
def _k5_compact(graph_hbm, t_hbm, need_hbm,
                gtv_hbm, gti_hbm, eqv_hbm, eqi_hbm, ngt_hbm,
                Tv, needv, cbuf, gvb, gib, evb, eib, nsplat):
    b = lax.axis_index("s")
    c = lax.axis_index("c")
    pltpu.sync_copy(t_hbm, Tv)
    pltpu.sync_copy(need_hbm, needv)
    tb = plsc.load_gather(Tv, [_splat_i32(b)])
    nb = plsc.load_gather(needv, [_splat_i32(b)])

    def _zero(i, _):
        sl = pl.ds(i * 16, 16)
        gvb[sl] = jnp.zeros((16,), jnp.float32)
        gib[sl] = jnp.zeros((16,), jnp.int32)
        evb[sl] = jnp.zeros((16,), jnp.float32)
        eib[sl] = jnp.zeros((16,), jnp.int32)
        return 0
    lax.fori_loop(0, CBUF // 16, _zero, 0)

    iota16 = lax.broadcasted_iota(jnp.int32, (16,), 0)
    ebase = c * EH

    def _chunk(ci, carry):
        gb = pl.multiple_of(b * N_EDGES + ebase + ci * _K5CH, 8)
        pltpu.sync_copy(graph_hbm.at[pl.ds(gb, _K5CH)], cbuf)

        def _grp(k, cr):
            vs = [cbuf[pl.ds((k * 5 + i) * 16, 16)] for i in range(5)]
            pc = plsc.all_reduce_population_count(vs[0] >= tb)
            for i in range(1, 5):
                pc = pc + plsc.all_reduce_population_count(vs[i] >= tb)
            hits = pc[0]

            def _slow(cr2):
                for i in range(5):
                    v = vs[i]
                    ngt, mst, etot = cr2
                    m_gt = v > tb
                    m_eq = v == tb
                    gidx = (_splat_i32(ebase + ci * _K5CH + (k * 5 + i) * 16)
                            + iota16)
                    eqrank = plsc.cumsum(m_eq.astype(jnp.int32))
                    take = m_eq & ((_splat_i32(etot) + eqrank) <= nb)
                    plsc.store_compressed(gvb.at[pl.ds(ngt, 16)], v,
                                          mask=m_gt)
                    plsc.store_compressed(gib.at[pl.ds(ngt, 16)], gidx,
                                          mask=m_gt)
                    plsc.store_compressed(evb.at[pl.ds(mst, 16)], v,
                                          mask=take)
                    plsc.store_compressed(eib.at[pl.ds(mst, 16)], gidx,
                                          mask=take)
                    ngt = ngt + plsc.all_reduce_population_count(m_gt)[0]
                    mst = mst + plsc.all_reduce_population_count(take)[0]
                    etot = etot + plsc.all_reduce_population_count(m_eq)[0]
                    cr2 = (ngt, mst, etot)
                return cr2

            return lax.cond(hits > 0, _slow, lambda cr2: cr2, cr)
        return lax.fori_loop(0, _K5CH // 80, _grp, carry)

    ngt, mst, etot = lax.fori_loop(
        0, _K5NCH, _chunk, (jnp.int32(0), jnp.int32(0), jnp.int32(0)))
    nsplat[pl.ds(0, 16)] = _splat_i32(ngt)
    nsplat[pl.ds(16, 16)] = _splat_i32(etot)
    ob = pl.multiple_of((b * 2 + c) * CBUF, 8)
    pltpu.sync_copy(gvb, gtv_hbm.at[pl.ds(ob, CBUF)])
    pltpu.sync_copy(gib, gti_hbm.at[pl.ds(ob, CBUF)])
    pltpu.sync_copy(evb, eqv_hbm.at[pl.ds(ob, CBUF)])
    pltpu.sync_copy(eib, eqi_hbm.at[pl.ds(ob, CBUF)])
    pltpu.sync_copy(nsplat,
                    ngt_hbm.at[pl.ds(pl.multiple_of((b * 2 + c) * 32, 8), 32)])


def _k6_body(vrow_ref, vcol_ref, irow_ref, icol_ref, ngt_ref, delta_ref,
             w_ref, if_ref):
    vrow = vrow_ref[0]
    vcol = vcol_ref[0]
    irow = irow_ref[0]
    icol = icol_ref[0]
    cnts = ngt_ref[0]
    n0 = jnp.max(cnts[:, 0:16])
    e0 = jnp.max(cnts[:, 16:32])
    n1 = jnp.max(cnts[:, 32:48])
    e1 = jnp.max(cnts[:, 48:64])
    need = K_TOP - n0 - n1
    m0 = jnp.minimum(need, e0)
    m1 = need - e0

    one = jnp.int32(1)
    zero = jnp.int32(0)

    def _valid(pos):
        r0 = jnp.where(pos < n0, one, zero)
        r1 = jnp.where(pos - CBUF < n1, one, zero)
        r2 = jnp.where(pos - 2 * CBUF < m0, one, zero)
        r3 = jnp.where(pos - 3 * CBUF < m1, one, zero)
        lo = jnp.where(pos < CBUF, r0, r1)
        hi = jnp.where(pos < 3 * CBUF, r2, r3)
        return jnp.where(pos < 2 * CBUF, lo, hi)

    lane = lax.broadcasted_iota(jnp.int32, (1, NCAND), 1)
    valid_row = _valid(lane)
    sub = lax.broadcasted_iota(jnp.int32, (NCAND, 1), 0)
    valid_col = _valid(sub)

    gt_i = jnp.where(vrow > vcol, one, zero)
    eq_i = jnp.where(vrow == vcol, one, zero)
    lt_i = jnp.where(irow < icol, one, zero)
    beats = (gt_i + eq_i * lt_i) * valid_row
    pos = jnp.sum(beats, axis=1, keepdims=True)

    kk = lax.broadcasted_iota(jnp.int32, (1, K_TOP), 1)
    oh = jnp.where(pos == kk, one, zero) * valid_col
    ohf = oh.astype(jnp.float32)
    w = jnp.sum(vcol * ohf, axis=0, keepdims=True)
    idx = jnp.sum(icol * ohf, axis=0, keepdims=True)
    w_ref[0] = w + delta_ref[0, 0, 0]
    if_ref[0] = idx


_k6_sort = pl.pallas_call(
    _k6_body,
    grid=(B,),
    in_specs=[
        pl.BlockSpec((1, 1, NCAND), lambda b: (b, 0, 0)),
        pl.BlockSpec((1, NCAND, 1), lambda b: (b, 0, 0)),
        pl.BlockSpec((1, 1, NCAND), lambda b: (b, 0, 0)),
        pl.BlockSpec((1, NCAND, 1), lambda b: (b, 0, 0)),
        pl.BlockSpec((1, 1, 64), lambda b: (b, 0, 0)),
        pl.BlockSpec((1, 1, 1), lambda b: (0, 0, 0)),
    ],
    out_specs=(
        pl.BlockSpec((1, 1, K_TOP), lambda b: (b, 0, 0)),
        pl.BlockSpec((1, 1, K_TOP), lambda b: (b, 0, 0)),
    ),
    out_shape=(
        jax.ShapeDtypeStruct((B, 1, K_TOP), jnp.float32),
        jax.ShapeDtypeStruct((B, 1, K_TOP), jnp.float32),
    ),
)


_ROWS_PER_W = (B * K_TOP) // 32


@functools.partial(
    pl.kernel,
    out_type=jax.ShapeDtypeStruct((B * K_TOP, D_FEAT), jnp.float32),
    mesh=_mesh,
    compiler_params=_sc_params,
    scratch_types=[
        pltpu.VMEM((_ROWS_PER_W,), jnp.int32),
        pltpu.VMEM((_ROWS_PER_W,), jnp.int32),
        pltpu.VMEM((_ROWS_PER_W,), jnp.float32),
        pltpu.VMEM((_ROWS_PER_W, D_FEAT), jnp.float32),
        pltpu.SemaphoreType.DMA,
    ],
)
def _k7_gather(topidx_hbm, topw_hbm, src_hbm, emb_hbm, out_hbm,
               ev, nv, wv, rows, sem):
    b = lax.axis_index("s")
    c = lax.axis_index("c")
    w = b * 2 + c
    base = pl.multiple_of(w * _ROWS_PER_W, 8)
    pltpu.sync_copy(topidx_hbm.at[pl.ds(base, _ROWS_PER_W)], ev)
    pltpu.sync_copy(topw_hbm.at[pl.ds(base, _ROWS_PER_W)], wv)
    pltpu.async_copy(src_hbm.at[ev], nv, sem).wait()
    pltpu.async_copy(emb_hbm.at[nv], rows, sem).wait()

    def _scale_row(i, _):
        ws = plsc.load_gather(wv, [_splat_i32(i)])
        for ch in range(D_FEAT // 16):
            sl = pl.ds(ch * 16, 16)
            rows[i, sl] = rows[i, sl] * ws
        return 0
    lax.fori_loop(0, _ROWS_PER_W, _scale_row, 0)
    pltpu.sync_copy(rows, out_hbm.at[pl.ds(base, _ROWS_PER_W)])


def kernel(list_questions, attention_question, edge_index, num_max_nodes,
           init_graph_tensor, node_embedding, w_importance):
    f32 = jnp.float32
    i32 = jnp.int32
    src = edge_index[0].astype(i32)
    dst = edge_index[1].astype(i32)

    qpad = jnp.zeros((B, QP), i32).at[:, :Q].set(
        list_questions.astype(i32)).reshape(B * QP)
    apad = jnp.zeros((B, QP), f32).at[:, :Q].set(
        attention_question).reshape(B * QP)
    wpad = jnp.zeros((QP,), f32).at[:Q].set(w_importance)

    graph = _k23_graph(qpad, apad, wpad, src, dst, init_graph_tensor)
    t_b, ngt_b, need_b = _k4_threshold(graph.reshape(B, N_EDGES))
    gtv, gti, eqv, eqi, ngt = _k5_compact(
        graph, t_b.reshape(B), need_b.reshape(B))
    gtv = gtv.reshape(B, 2 * CBUF)
    gti = gti.reshape(B, 2 * CBUF)
    eqv = eqv.reshape(B, 2 * CBUF)
    eqi = eqi.reshape(B, 2 * CBUF)
    ngt = ngt.reshape(B, 64)

    cat_v = jnp.concatenate([gtv, eqv], axis=1)
    cat_i = jnp.concatenate([gti, eqi], axis=1).astype(f32)
    delta = jnp.asarray(num_max_nodes - K_TOP, f32).reshape(1, 1, 1)
    top_w, top_if = _k6_sort(
        cat_v[:, None, :], cat_v[:, :, None],
        cat_i[:, None, :], cat_i[:, :, None],
        ngt[:, None, :], delta)

    top_idx = top_if.reshape(B * K_TOP).astype(i32)
    out = _k7_gather(top_idx, top_w.reshape(B * K_TOP), src, node_embedding)
    return out.reshape(B, K_TOP, D_FEAT)


# speedup vs baseline: 1.5530x; 1.0279x over previous
import functools

import jax
import jax.numpy as jnp
from jax import lax
from jax.experimental import pallas as pl
from jax.experimental.pallas import tpu as pltpu
from jax.experimental.pallas import tpu_sc as plsc

N_NODES = 10000
N_EDGES = 160000
B = 16
Q = 20
K_TOP = 256
D_FEAT = 256
PROP_THRESHOLD = 0.5
ATTEN_COEF = 0.25

QP = 32
EH = N_EDGES // 2
CH = 4000
NCHUNK = EH // CH
CBUF = 272
NCAND = 4 * CBUF

_mesh = plsc.VectorSubcoreMesh(core_axis_name="c", subcore_axis_name="s")
_sc_params = pltpu.CompilerParams(needs_layout_passes=False)


def _splat_i32(x):
    return jnp.zeros((16,), jnp.int32) + x


def _splat_f32(x):
    return jnp.zeros((16,), jnp.float32) + x


@functools.partial(
    pl.kernel,
    out_type=jax.ShapeDtypeStruct((B * N_EDGES,), jnp.float32),
    mesh=_mesh,
    compiler_params=_sc_params,
    scratch_types=[
        pltpu.VMEM((QP,), jnp.int32),
        pltpu.VMEM((QP,), jnp.float32),
        pltpu.VMEM((QP,), jnp.float32),
        pltpu.VMEM((QP,), jnp.float32),
        pltpu.VMEM((N_NODES,), jnp.float32),
        pltpu.VMEM((N_NODES,), jnp.float32),
        pltpu.VMEM((CH,), jnp.int32),
        pltpu.VMEM((CH,), jnp.int32),
        pltpu.VMEM((CH,), jnp.float32),
        pltpu.VMEM((CH,), jnp.int32),
        pltpu.VMEM((CH,), jnp.int32),
        pltpu.VMEM((CH,), jnp.float32),
        pltpu.VMEM((CH,), jnp.float32),
        pltpu.SemaphoreType.DMA,
        pltpu.SemaphoreType.DMA,
    ],
)
def _k23_graph(q_hbm, a_hbm, w_hbm, src_hbm, dst_hbm, init_hbm, graph_hbm,
               qv, av, wv, gv, Ct, At, sbuf0, dbuf0, ibuf0,
               sbuf1, dbuf1, ibuf1, obuf, sem0, sem1):
    b = lax.axis_index("s")
    c = lax.axis_index("c")
    pltpu.sync_copy(q_hbm.at[pl.ds(pl.multiple_of(b * QP, 8), QP)], qv)
    pltpu.sync_copy(a_hbm.at[pl.ds(pl.multiple_of(b * QP, 8), QP)], av)
    pltpu.sync_copy(w_hbm, wv)

    for t in range(2):
        sl = pl.ds(t * 16, 16)
        x = av[sl] * wv[sl]
        imp = 1.0 / (1.0 + jnp.exp(-x))
        g = jnp.where(imp >= PROP_THRESHOLD, imp, 0.0)
        if t == 1:
            lane = lax.broadcasted_iota(jnp.int32, (16,), 0)
            g = jnp.where(lane < (Q - 16), g, 0.0)
        gv[sl] = g

    def _zero(i, _):
        Ct[pl.ds(i * 16, 16)] = jnp.zeros((16,), jnp.float32)
        return 0
    lax.fori_loop(0, N_NODES // 16, _zero, 0)

    for t in range(2):
        sl = pl.ds(t * 16, 16)
        plsc.addupdate_scatter(Ct, [qv[sl]], gv[sl])

    atts = []
    n_att = jnp.int32(0)
    for t in range(2):
        sl = pl.ds(t * 16, 16)
        qch = qv[sl]
        v = jnp.zeros((16,), jnp.float32)
        for j in range(Q):
            qs = plsc.load_gather(qv, [jnp.full((16,), j, jnp.int32)])
            gs = plsc.load_gather(gv, [jnp.full((16,), j, jnp.int32)])
            v = v + jnp.where(qch == qs, gs, 0.0)
        a = ATTEN_COEF * v
        a = jnp.where(a >= PROP_THRESHOLD, a, 0.0)
        atts.append(a)
        n_att = n_att + jnp.max(plsc.all_reduce_population_count(a > 0.0))

    @pl.when(n_att > 0)
    def _slow_hop():
        def _zeroA(i, _):
            At[pl.ds(i * 16, 16)] = jnp.zeros((16,), jnp.float32)
            return 0
        lax.fori_loop(0, N_NODES // 16, _zeroA, 0)
        for t in range(2):
            plsc.store_scatter(At, [qv[pl.ds(t * 16, 16)]], atts[t])

        def _chunk(ci, _):
            base = pl.multiple_of(ci * CH, 8)
            pltpu.sync_copy(src_hbm.at[pl.ds(base, CH)], sbuf0)
            pltpu.sync_copy(dst_hbm.at[pl.ds(base, CH)], dbuf0)

            def _vec(k, __):
                sl = pl.ds(k * 16, 16)
                sv = sbuf0[sl]
                dv = dbuf0[sl]
                a_s = plsc.load_gather(At, [sv])
                a_d = plsc.load_gather(At, [dv])
                plsc.addupdate_scatter(Ct, [dv], a_s)
                plsc.addupdate_scatter(Ct, [sv], a_d)
                return 0
            lax.fori_loop(0, CH // 16, _vec, 0)
            return 0
        lax.fori_loop(0, N_EDGES // CH, _chunk, 0)

    bufs = ((sbuf0, dbuf0, ibuf0, sem0), (sbuf1, dbuf1, ibuf1, sem1))

    def _in_slices(ci):
        base = pl.multiple_of(c * EH + ci * CH, 8)
        return (src_hbm.at[pl.ds(base, CH)], dst_hbm.at[pl.ds(base, CH)],
                init_hbm.at[pl.ds(base, CH)])

    def _start_in(ci, bs):
        s0, s1, s2 = _in_slices(ci)
        pltpu.async_copy(s0, bs[0], bs[3])
        pltpu.async_copy(s1, bs[1], bs[3])
        pltpu.async_copy(s2, bs[2], bs[3])

    def _wait_in(ci, bs):
        s0, s1, s2 = _in_slices(ci)
        pltpu.make_async_copy(s0, bs[0], bs[3]).wait()
        pltpu.make_async_copy(s1, bs[1], bs[3]).wait()
        pltpu.make_async_copy(s2, bs[2], bs[3]).wait()

    def _compute(ci, bs):
        sb, db, ib = bs[0], bs[1], bs[2]

        def _vec(k, __):
            for u in range(2):
                sl = pl.ds((k * 2 + u) * 16, 16)
                cs = plsc.load_gather(Ct, [sb[sl]])
                cd = plsc.load_gather(Ct, [db[sl]])
                obuf[sl] = ib[sl] + cs + cd
            return 0
        lax.fori_loop(0, CH // 32, _vec, 0)
        gbase = pl.multiple_of(b * N_EDGES + c * EH + ci * CH, 8)
        pltpu.sync_copy(obuf, graph_hbm.at[pl.ds(gbase, CH)])

    _start_in(jnp.int32(0), bufs[0])

    def _pair(pi, _):
        ci0 = pi * 2
        _wait_in(ci0, bufs[0])
        _start_in(ci0 + 1, bufs[1])
        _compute(ci0, bufs[0])
        _wait_in(ci0 + 1, bufs[1])

        @pl.when(pi < NCHUNK // 2 - 1)
        def _():
            _start_in(ci0 + 2, bufs[0])
        _compute(ci0 + 1, bufs[1])
        return 0
    lax.fori_loop(0, NCHUNK // 2, _pair, 0)


_CB = 80000
_NB = N_EDGES // _CB


def _k4_body(graph_ref, T_ref, ngt_ref, need_ref):
    def count_ge(test):
        def blk(k, acc):
            v = graph_ref[:, pl.ds(k * _CB, _CB)]
            bv = lax.bitcast_convert_type(v, jnp.int32)
            return acc + jnp.sum((bv >= test).astype(jnp.int32), axis=1,
                                 keepdims=True)
        return lax.fori_loop(0, _NB, blk, jnp.zeros((B, 1), jnp.int32))

    cand = jnp.where(count_ge(jnp.full((B, 1), 1 << 30, jnp.int32)) >= K_TOP,
                     jnp.full((B, 1), 1 << 30, jnp.int32),
                     jnp.zeros((B, 1), jnp.int32))

    def radix_step(i, cand):
        shift = 28 - 2 * i
        t1 = cand | (1 << shift)
        t2 = cand | (2 << shift)
        t3 = cand | (3 << shift)

        def blk(k, accs):
            a1, a2, a3 = accs
            v = graph_ref[:, pl.ds(k * _CB, _CB)]
            bv = lax.bitcast_convert_type(v, jnp.int32)
            a1 = a1 + jnp.sum((bv >= t1).astype(jnp.int32), axis=1,
                              keepdims=True)
            a2 = a2 + jnp.sum((bv >= t2).astype(jnp.int32), axis=1,
                              keepdims=True)
            a3 = a3 + jnp.sum((bv >= t3).astype(jnp.int32), axis=1,
                              keepdims=True)
            return (a1, a2, a3)
        z = jnp.zeros((B, 1), jnp.int32)
        c1, c2, c3 = lax.fori_loop(0, _NB, blk, (z, z, z))
        return jnp.where(
            c3 >= K_TOP, t3,
            jnp.where(c2 >= K_TOP, t2, jnp.where(c1 >= K_TOP, t1, cand)))

    cand = lax.fori_loop(0, 15, radix_step, cand)
    ngt = count_ge(cand + 1)
    T_ref[...] = lax.bitcast_convert_type(cand, jnp.float32)
    ngt_ref[...] = ngt
    need_ref[...] = K_TOP - ngt


_k4_threshold = pl.pallas_call(
    _k4_body,
    out_shape=(
        jax.ShapeDtypeStruct((B, 1), jnp.float32),
        jax.ShapeDtypeStruct((B, 1), jnp.int32),
        jax.ShapeDtypeStruct((B, 1), jnp.int32),
    ),
)


_K5CH = 4000
_K5NCH = EH // _K5CH


@functools.partial(
    pl.kernel,
    out_type=(
        jax.ShapeDtypeStruct((B * 2 * CBUF,), jnp.float32),
        jax.ShapeDtypeStruct((B * 2 * CBUF,), jnp.int32),
        jax.ShapeDtypeStruct((B * 2 * CBUF,), jnp.float32),
        jax.ShapeDtypeStruct((B * 2 * CBUF,), jnp.int32),
        jax.ShapeDtypeStruct((B * 2 * 2 * 16,), jnp.int32),
    ),
    mesh=_mesh,
    compiler_params=_sc_params,
    scratch_types=[
        pltpu.VMEM((16,), jnp.float32),
        pltpu.VMEM((16,), jnp.int32),
        pltpu.VMEM((_K5CH,), jnp.float32),
        pltpu.VMEM((CBUF,), jnp.float32),
        pltpu.VMEM((CBUF,), jnp.int32),
        pltpu.VMEM((CBUF,), jnp.float32),
        pltpu.VMEM((CBUF,), jnp.int32),
        pltpu.VMEM((32,), jnp.int32),
    ],
)
def _k5_compact(graph_hbm, t_hbm, need_hbm,
                gtv_hbm, gti_hbm, eqv_hbm, eqi_hbm, ngt_hbm,
                Tv, needv, cbuf, gvb, gib, evb, eib, nsplat):
    b = lax.axis_index("s")
    c = lax.axis_index("c")
    pltpu.sync_copy(t_hbm, Tv)
    pltpu.sync_copy(need_hbm, needv)
    tb = plsc.load_gather(Tv, [_splat_i32(b)])
    nb = plsc.load_gather(needv, [_splat_i32(b)])

    def _zero(i, _):
        sl = pl.ds(i * 16, 16)
        gvb[sl] = jnp.zeros((16,), jnp.float32)
        gib[sl] = jnp.zeros((16,), jnp.int32)
        evb[sl] = jnp.zeros((16,), jnp.float32)
        eib[sl] = jnp.zeros((16,), jnp.int32)
        return 0
    lax.fori_loop(0, CBUF // 16, _zero, 0)

    iota16 = lax.broadcasted_iota(jnp.int32, (16,), 0)
    ebase = c * EH

    def _chunk(ci, carry):
        gb = pl.multiple_of(b * N_EDGES + ebase + ci * _K5CH, 8)
        pltpu.sync_copy(graph_hbm.at[pl.ds(gb, _K5CH)], cbuf)

        def _grp(k, cr):
            vs = [cbuf[pl.ds((k * 5 + i) * 16, 16)] for i in range(5)]
            pc = plsc.all_reduce_population_count(vs[0] >= tb)
            for i in range(1, 5):
                pc = pc + plsc.all_reduce_population_count(vs[i] >= tb)
            hits = pc[0]

            def _slow(cr2):
                for i in range(5):
                    v = vs[i]
                    ngt, mst, etot = cr2
                    m_gt = v > tb
                    m_eq = v == tb
                    gidx = (_splat_i32(ebase + ci * _K5CH + (k * 5 + i) * 16)
                            + iota16)
                    eqrank = plsc.cumsum(m_eq.astype(jnp.int32))
                    take = m_eq & ((_splat_i32(etot) + eqrank) <= nb)
                    plsc.store_compressed(gvb.at[pl.ds(ngt, 16)], v,
                                          mask=m_gt)
                    plsc.store_compressed(gib.at[pl.ds(ngt, 16)], gidx,
                                          mask=m_gt)
                    plsc.store_compressed(evb.at[pl.ds(mst, 16)], v,
                                          mask=take)
                    plsc.store_compressed(eib.at[pl.ds(mst, 16)], gidx,
                                          mask=take)
                    ngt = ngt + plsc.all_reduce_population_count(m_gt)[0]
                    mst = mst + plsc.all_reduce_population_count(take)[0]
                    etot = etot + plsc.all_reduce_population_count(m_eq)[0]
                    cr2 = (ngt, mst, etot)
                return cr2

            return lax.cond(hits > 0, _slow, lambda cr2: cr2, cr)
        return lax.fori_loop(0, _K5CH // 80, _grp, carry)

    ngt, mst, etot = lax.fori_loop(
        0, _K5NCH, _chunk, (jnp.int32(0), jnp.int32(0), jnp.int32(0)))
    nsplat[pl.ds(0, 16)] = _splat_i32(ngt)
    nsplat[pl.ds(16, 16)] = _splat_i32(etot)
    ob = pl.multiple_of((b * 2 + c) * CBUF, 8)
    pltpu.sync_copy(gvb, gtv_hbm.at[pl.ds(ob, CBUF)])
    pltpu.sync_copy(gib, gti_hbm.at[pl.ds(ob, CBUF)])
    pltpu.sync_copy(evb, eqv_hbm.at[pl.ds(ob, CBUF)])
    pltpu.sync_copy(eib, eqi_hbm.at[pl.ds(ob, CBUF)])
    pltpu.sync_copy(nsplat,
                    ngt_hbm.at[pl.ds(pl.multiple_of((b * 2 + c) * 32, 8), 32)])


def _k6_body(vrow_ref, vcol_ref, irow_ref, icol_ref, ngt_ref, delta_ref,
             w_ref, if_ref):
    vrow = vrow_ref[0]
    vcol = vcol_ref[0]
    irow = irow_ref[0]
    icol = icol_ref[0]
    cnts = ngt_ref[0]
    n0 = jnp.max(cnts[:, 0:16])
    e0 = jnp.max(cnts[:, 16:32])
    n1 = jnp.max(cnts[:, 32:48])
    e1 = jnp.max(cnts[:, 48:64])
    need = K_TOP - n0 - n1
    m0 = jnp.minimum(need, e0)
    m1 = need - e0

    one = jnp.int32(1)
    zero = jnp.int32(0)

    def _valid(pos):
        r0 = jnp.where(pos < n0, one, zero)
        r1 = jnp.where(pos - CBUF < n1, one, zero)
        r2 = jnp.where(pos - 2 * CBUF < m0, one, zero)
        r3 = jnp.where(pos - 3 * CBUF < m1, one, zero)
        lo = jnp.where(pos < CBUF, r0, r1)
        hi = jnp.where(pos < 3 * CBUF, r2, r3)
        return jnp.where(pos < 2 * CBUF, lo, hi)

    lane = lax.broadcasted_iota(jnp.int32, (1, NCAND), 1)
    valid_row = _valid(lane)
    sub = lax.broadcasted_iota(jnp.int32, (NCAND, 1), 0)
    valid_col = _valid(sub)

    gt_i = jnp.where(vrow > vcol, one, zero)
    eq_i = jnp.where(vrow == vcol, one, zero)
    lt_i = jnp.where(irow < icol, one, zero)
    beats = (gt_i + eq_i * lt_i) * valid_row
    pos = jnp.sum(beats, axis=1, keepdims=True)

    kk = lax.broadcasted_iota(jnp.int32, (1, K_TOP), 1)
    oh = jnp.where(pos == kk, one, zero) * valid_col
    ohf = oh.astype(jnp.float32)
    w = jnp.sum(vcol * ohf, axis=0, keepdims=True)
    idx = jnp.sum(icol * ohf, axis=0, keepdims=True)
    w_ref[0] = w + delta_ref[0, 0, 0]
    if_ref[0] = idx


_k6_sort = pl.pallas_call(
    _k6_body,
    grid=(B,),
    in_specs=[
        pl.BlockSpec((1, 1, NCAND), lambda b: (b, 0, 0)),
        pl.BlockSpec((1, NCAND, 1), lambda b: (b, 0, 0)),
        pl.BlockSpec((1, 1, NCAND), lambda b: (b, 0, 0)),
        pl.BlockSpec((1, NCAND, 1), lambda b: (b, 0, 0)),
        pl.BlockSpec((1, 1, 64), lambda b: (b, 0, 0)),
        pl.BlockSpec((1, 1, 1), lambda b: (0, 0, 0)),
    ],
    out_specs=(
        pl.BlockSpec((1, 1, K_TOP), lambda b: (b, 0, 0)),
        pl.BlockSpec((1, 1, K_TOP), lambda b: (b, 0, 0)),
    ),
    out_shape=(
        jax.ShapeDtypeStruct((B, 1, K_TOP), jnp.float32),
        jax.ShapeDtypeStruct((B, 1, K_TOP), jnp.float32),
    ),
)


_ROWS_PER_W = (B * K_TOP) // 32


@functools.partial(
    pl.kernel,
    out_type=jax.ShapeDtypeStruct((B * K_TOP, D_FEAT), jnp.float32),
    mesh=_mesh,
    compiler_params=_sc_params,
    scratch_types=[
        pltpu.VMEM((_ROWS_PER_W,), jnp.int32),
        pltpu.VMEM((_ROWS_PER_W,), jnp.int32),
        pltpu.VMEM((_ROWS_PER_W,), jnp.float32),
        pltpu.VMEM((_ROWS_PER_W, D_FEAT), jnp.float32),
        pltpu.SemaphoreType.DMA,
    ],
)
def _k7_gather(topidx_hbm, topw_hbm, src_hbm, emb_hbm, out_hbm,
               ev, nv, wv, rows, sem):
    b = lax.axis_index("s")
    c = lax.axis_index("c")
    w = b * 2 + c
    base = pl.multiple_of(w * _ROWS_PER_W, 8)
    pltpu.sync_copy(topidx_hbm.at[pl.ds(base, _ROWS_PER_W)], ev)
    pltpu.sync_copy(topw_hbm.at[pl.ds(base, _ROWS_PER_W)], wv)
    pltpu.async_copy(src_hbm.at[ev], nv, sem).wait()
    pltpu.async_copy(emb_hbm.at[nv], rows, sem).wait()

    def _scale_row(i, _):
        ws = plsc.load_gather(wv, [_splat_i32(i)])
        for ch in range(D_FEAT // 16):
            sl = pl.ds(ch * 16, 16)
            rows[i, sl] = rows[i, sl] * ws
        return 0
    lax.fori_loop(0, _ROWS_PER_W, _scale_row, 0)
    pltpu.sync_copy(rows, out_hbm.at[pl.ds(base, _ROWS_PER_W)])


def kernel(list_questions, attention_question, edge_index, num_max_nodes,
           init_graph_tensor, node_embedding, w_importance):
    f32 = jnp.float32
    i32 = jnp.int32
    src = edge_index[0].astype(i32)
    dst = edge_index[1].astype(i32)

    qpad = jnp.zeros((B, QP), i32).at[:, :Q].set(
        list_questions.astype(i32)).reshape(B * QP)
    apad = jnp.zeros((B, QP), f32).at[:, :Q].set(
        attention_question).reshape(B * QP)
    wpad = jnp.zeros((QP,), f32).at[:Q].set(w_importance)

    graph = _k23_graph(qpad, apad, wpad, src, dst, init_graph_tensor)
    t_b, ngt_b, need_b = _k4_threshold(graph.reshape(B, N_EDGES))
    gtv, gti, eqv, eqi, ngt = _k5_compact(
        graph, t_b.reshape(B), need_b.reshape(B))
    gtv = gtv.reshape(B, 2 * CBUF)
    gti = gti.reshape(B, 2 * CBUF)
    eqv = eqv.reshape(B, 2 * CBUF)
    eqi = eqi.reshape(B, 2 * CBUF)
    ngt = ngt.reshape(B, 64)

    cat_v = jnp.concatenate([gtv, eqv], axis=1)
    cat_i = jnp.concatenate([gti, eqi], axis=1).astype(f32)
    delta = jnp.asarray(num_max_nodes - K_TOP, f32).reshape(1, 1, 1)
    top_w, top_if = _k6_sort(
        cat_v[:, None, :], cat_v[:, :, None],
        cat_i[:, None, :], cat_i[:, :, None],
        ngt[:, None, :], delta)

    top_idx = top_if.reshape(B * K_TOP).astype(i32)
    out = _k7_gather(top_idx, top_w.reshape(B * K_TOP), src, node_embedding)
    return out.reshape(B, K_TOP, D_FEAT)
